# baseline (speedup 1.0000x reference)
import functools

import jax
import jax.numpy as jnp
from jax import lax
from jax.experimental import pallas as pl
from jax.experimental.pallas import tpu as pltpu
from jax.experimental.pallas import tpu_sc as plsc

F32 = jnp.float32
CH = 125
KG = 4


def _relu(t):
    return jnp.maximum(t, 0.0)


def _mm(a, b):
    return jnp.dot(a, b, preferred_element_type=F32)


def _sc_mesh():
    return plsc.VectorSubcoreMesh(core_axis_name="c", subcore_axis_name="s")


@functools.partial(jax.jit, static_argnums=())
def _sc_gather_pair(table, idx_row, idx_col):
    n, d = table.shape
    e = idx_row.shape[0]
    info = plsc.get_sparse_core_info()
    nc, ns = info.num_cores, info.num_subcores
    nw = nc * ns
    ch = CH
    pw = e // (nw * ch)
    ng = pw // KG
    grp = KG * ch
    idx_row3 = idx_row.reshape(nw, pw, ch)
    idx_col3 = idx_col.reshape(nw, pw, ch)

    @functools.partial(
        pl.kernel,
        out_type=(jax.ShapeDtypeStruct((e, d), F32),
                  jax.ShapeDtypeStruct((e, d), F32)),
        mesh=_sc_mesh(),
        scratch_types=[
            pltpu.VMEM((pw, ch), jnp.int32),
            pltpu.VMEM((2, grp, d), F32),
            pltpu.SemaphoreType.DMA,
            pltpu.SemaphoreType.DMA((2,)),
        ],
        compiler_params=pltpu.CompilerParams(use_tc_tiling_on_sc=False),
    )
    def k(table_h, ir_h, ic_h, or_h, oc_h, idx_v, buf_v, gsem, osem):
        cid = lax.axis_index("c")
        sid = lax.axis_index("s")
        wid = sid * nc + cid
        rbase = wid * pw * ch

        def run(idx_h, out_h):
            pltpu.sync_copy(idx_h.at[wid], idx_v)

            def body(g, carry):
                par = lax.rem(g, 2)
                @pl.when(g >= 2)
                def _():
                    pltpu.make_async_copy(
                        buf_v.at[par],
                        out_h.at[pl.ds(rbase + g * grp, grp)],
                        osem.at[par]).wait()

                for kk in range(KG):
                    pltpu.async_copy(
                        table_h.at[idx_v.at[g * KG + kk]],
                        buf_v.at[par, pl.ds(kk * ch, ch)], gsem)
                for kk in range(KG):
                    pltpu.make_async_copy(
                        table_h.at[idx_v.at[g * KG + kk]],
                        buf_v.at[par, pl.ds(kk * ch, ch)], gsem).wait()
                pltpu.async_copy(
                    buf_v.at[par],
                    out_h.at[pl.ds(rbase + g * grp, grp)], osem.at[par])
                return carry

            lax.fori_loop(0, ng, body, 0)
            for par in range(2):
                pltpu.make_async_copy(
                    buf_v.at[par],
                    out_h.at[pl.ds(rbase, grp)], osem.at[par]).wait()

        run(ir_h, or_h)
        run(ic_h, oc_h)

    return k(table, idx_row3, idx_col3)


def _sc_scatter_add(vals, idx, zeros):
    e, w = vals.shape
    n = zeros.shape[0]
    info = plsc.get_sparse_core_info()
    nc, ns = info.num_cores, info.num_subcores
    nw = nc * ns
    ch = CH
    pw = e // (nw * ch)
    ng = pw // KG
    grp = KG * ch
    idx3 = idx.reshape(nw, pw, ch)
    rows_per = (n // ns) // 8 * 8
    rows_last = n - rows_per * (ns - 1)

    @functools.partial(
        pl.kernel,
        out_type=(jax.ShapeDtypeStruct((n, w), F32),
                  jax.ShapeDtypeStruct((n, w), F32)),
        mesh=_sc_mesh(),
        scratch_types=[
            pltpu.VMEM((pw, ch), jnp.int32),
            pltpu.VMEM((2, grp, w), F32),
            pltpu.VMEM_SHARED((n, w), F32),
            pltpu.SemaphoreType.DMA((2,)),
            pltpu.SemaphoreType.DMA((2,)),
        ],
        compiler_params=pltpu.CompilerParams(use_tc_tiling_on_sc=False),
    )
    def k(vals_h, idx_h, zeros_h, p0_h, p1_h, idx_v, buf_v, acc, lsem, ssem):
        cid = lax.axis_index("c")
        sid = lax.axis_index("s")
        wid = sid * nc + cid
        rbase = wid * pw * ch

        pltpu.sync_copy(idx_h.at[wid], idx_v)

        def init_and_out(fn):
            @pl.when(sid < ns - 1)
            def _():
                fn(pl.ds(sid * rows_per, rows_per))

            @pl.when(sid == ns - 1)
            def _():
                fn(pl.ds((ns - 1) * rows_per, rows_last))

        init_and_out(lambda sl: pltpu.sync_copy(zeros_h.at[sl], acc.at[sl]))
        plsc.subcore_barrier()

        def load_grp(g, par):
            pltpu.async_copy(vals_h.at[pl.ds(rbase + g * grp, grp)],
                             buf_v.at[par], lsem.at[par])

        def drain_adds(g, par):
            for kk in range(KG):
                pltpu.make_async_copy(
                    buf_v.at[par, pl.ds(kk * ch, ch)],
                    acc.at[idx_v.at[g * KG + kk]], ssem.at[par]).wait()

        load_grp(0, 0)

        def body(g, carry):
            par = lax.rem(g, 2)
            pltpu.make_async_copy(vals_h.at[pl.ds(rbase + g * grp, grp)],
                                  buf_v.at[par], lsem.at[par]).wait()

            @pl.when(g >= 1)
            def _():
                drain_adds(g - 1, 1 - par)

            @pl.when(g + 1 < ng)
            def _():
                load_grp(g + 1, 1 - par)

            for kk in range(KG):
                pltpu.async_copy(
                    buf_v.at[par, pl.ds(kk * ch, ch)],
                    acc.at[idx_v.at[g * KG + kk]], ssem.at[par], add=True)
            return carry

        lax.fori_loop(0, ng, body, 0)
        drain_adds(ng - 1, (ng - 1) % 2)
        plsc.subcore_barrier()

        @pl.when(cid == 0)
        def _():
            init_and_out(lambda sl: pltpu.sync_copy(acc.at[sl], p0_h.at[sl]))

        @pl.when(cid == 1)
        def _():
            init_and_out(lambda sl: pltpu.sync_copy(acc.at[sl], p1_h.at[sl]))

    return k(vals, idx3, zeros)


def _tc_embed(x, w, b):
    n, din = x.shape
    dout = w.shape[1]
    blk = n // 5

    def body(x_ref, w_ref, b_ref, o_ref):
        o_ref[...] = _mm(x_ref[...], w_ref[...]) + b_ref[...]

    return pl.pallas_call(
        body,
        grid=(n // blk,),
        in_specs=[
            pl.BlockSpec((blk, din), lambda i: (i, 0)),
            pl.BlockSpec((din, dout), lambda i: (0, 0)),
            pl.BlockSpec((1, dout), lambda i: (0, 0)),
        ],
        out_specs=pl.BlockSpec((blk, dout), lambda i: (i, 0)),
        out_shape=jax.ShapeDtypeStruct((n, dout), F32),
    )(x, w, b)


def _tc_add3(h, p0, p1):
    n, d = h.shape
    blk = n // 5

    def body(a_ref, b_ref, c_ref, o_ref):
        o_ref[...] = a_ref[...] + b_ref[...] + c_ref[...]

    return pl.pallas_call(
        body,
        grid=(n // blk,),
        in_specs=[pl.BlockSpec((blk, d), lambda i: (i, 0))] * 3,
        out_specs=pl.BlockSpec((blk, d), lambda i: (i, 0)),
        out_shape=jax.ShapeDtypeStruct((n, d), F32),
    )(h, p0, p1)


def _tc_edge_mlp(hr, hc, e_in, ws, first, last):
    e_cnt = hr.shape[0]
    blk = 2000
    grid = e_cnt // blk
    ein_w = e_in.shape[1]
    eout_w = 48 if last else 32

    names = (["wee", "bee"] if first else []) + [
        "wn0h", "wn0e", "bn0", "wn1", "bn1", "wn2", "bn2",
        "we0h", "we0c", "we0e", "be0", "we1", "be1",
    ]
    warrs = [ws[nm] for nm in names]

    bf16 = jnp.bfloat16

    def body(hr_ref, hc_ref, e_ref, *rest):
        wr = {nm: r[...] for nm, r in zip(names, rest[:len(names)])}
        wb = {nm: v.astype(bf16) for nm, v in wr.items()}
        dh_ref, eo_ref = rest[len(names):]
        hr_b = hr_ref[...].astype(bf16)
        hc_b = hc_ref[...].astype(bf16)
        if first:
            e_b = _mm(e_ref[...].astype(bf16), wb["wee"]) + wr["bee"]
        else:
            e_b = e_ref[...]
        e_b16 = e_b.astype(bf16)
        d = _relu(_mm(hr_b, wb["wn0h"]) + _mm(e_b16, wb["wn0e"]) + wr["bn0"])
        d = _relu(_mm(d.astype(bf16), wb["wn1"]) + wr["bn1"])
        dh_ref[...] = _mm(d.astype(bf16), wb["wn2"]) + wr["bn2"]
        u = _relu(_mm(hr_b, wb["we0h"]) + _mm(hc_b, wb["we0c"])
                  + _mm(e_b16, wb["we0e"]) + wr["be0"])
        e_new = e_b + _mm(u.astype(bf16), wb["we1"]) + wr["be1"]
        if last:
            eo_ref[...] = jnp.concatenate(
                [e_new, jnp.ones((blk, 1), F32), jnp.zeros((blk, 15), F32)],
                axis=1)
        else:
            eo_ref[...] = e_new

    in_specs = [
        pl.BlockSpec((blk, 64), lambda i: (i, 0)),
        pl.BlockSpec((blk, 64), lambda i: (i, 0)),
        pl.BlockSpec((blk, ein_w), lambda i: (i, 0)),
    ] + [pl.BlockSpec(w.shape, lambda i: (0, 0)) for w in warrs]

    return pl.pallas_call(
        body,
        grid=(grid,),
        in_specs=in_specs,
        out_specs=[
            pl.BlockSpec((blk, 64), lambda i: (i, 0)),
            pl.BlockSpec((blk, eout_w), lambda i: (i, 0)),
        ],
        out_shape=[
            jax.ShapeDtypeStruct((e_cnt, 64), F32),
            jax.ShapeDtypeStruct((e_cnt, eout_w), F32),
        ],
    )(hr, hc, e_in, *warrs)


def _tc_pool_head(h, p0, p1, ws):
    n = h.shape[0]
    blk = n // 5
    grid = n // blk
    names = ["wnp", "bnp", "wep", "bep", "wq", "bq", "query",
             "wk", "bk", "wv", "bv", "wo", "bo",
             "wh0", "bh0", "ln_g", "ln_b", "wh1", "bh1"]
    warrs = [ws[nm] for nm in names]

    heads, dh_ = 4, 64
    emb = heads * dh_

    def body(h_ref, p0_ref, p1_ref, *rest):
        wr = {nm: r[...] for nm, r in zip(names, rest[:len(names)])}
        out_ref = rest[len(names)]
        m_s, d_s, num_s = rest[len(names) + 1:]
        i = pl.program_id(0)

        @pl.when(i == 0)
        def _():
            m_s[...] = jnp.full((1, heads), -1e30, F32)
            d_s[...] = jnp.zeros((1, heads), F32)
            num_s[...] = jnp.zeros((heads, emb), F32)

        seg = p0_ref[...] + p1_ref[...]
        cnt = seg[:, 32:33]
        sums = _mm(seg[:, :32], wr["wep"]) + cnt * wr["bep"]
        hp = (_mm(h_ref[...], wr["wnp"]) + wr["bnp"]
              + sums / jnp.maximum(cnt, 1.0))
        kk = _mm(hp, wr["wk"]) + wr["bk"]
        vv = _mm(hp, wr["wv"]) + wr["bv"]

        q = _mm(wr["query"], wr["wq"]) + wr["bq"]
        colh = lax.broadcasted_iota(jnp.int32, (emb, heads), 0) // dh_
        rowh = lax.broadcasted_iota(jnp.int32, (emb, heads), 1)
        hsel = (colh == rowh).astype(F32)
        s = _mm(kk * q, hsel) * (1.0 / 8.0)

        m_prev = m_s[...]
        bm = jnp.max(s, axis=0, keepdims=True)
        m_new = jnp.maximum(m_prev, bm)
        corr = jnp.exp(m_prev - m_new)
        wgt = jnp.exp(s - m_new)
        d_s[...] = d_s[...] * corr + jnp.sum(wgt, axis=0, keepdims=True)
        num_s[...] = (num_s[...] * jnp.transpose(corr)
                      + lax.dot_general(wgt, vv, (((0,), (0,)), ((), ())),
                                        preferred_element_type=F32))
        m_s[...] = m_new

        @pl.when(i == grid - 1)
        def _():
            bd = jnp.transpose(hsel)
            o = jnp.sum(num_s[...] * bd, axis=0, keepdims=True)
            den = _mm(d_s[...], bd)
            o = o / den
            z = _relu(_mm(o, wr["wo"]) + wr["bo"])
            z = _relu(_mm(z, wr["wh0"]) + wr["bh0"])
            mu = jnp.mean(z, axis=-1, keepdims=True)
            var = jnp.mean((z - mu) ** 2, axis=-1, keepdims=True)
            zn = (z - mu) * lax.rsqrt(var + 1e-5)
            zn = zn * wr["ln_g"] + wr["ln_b"]
            out_ref[...] = _mm(zn, wr["wh1"]) + wr["bh1"]

    in_specs = [
        pl.BlockSpec((blk, 64), lambda i: (i, 0)),
        pl.BlockSpec((blk, 48), lambda i: (i, 0)),
        pl.BlockSpec((blk, 48), lambda i: (i, 0)),
    ] + [pl.BlockSpec(w.shape, lambda i: (0, 0)) for w in warrs]

    return pl.pallas_call(
        body,
        grid=(grid,),
        in_specs=in_specs,
        out_specs=pl.BlockSpec((1, 1024), lambda i: (0, 0)),
        out_shape=jax.ShapeDtypeStruct((1, 1024), F32),
        scratch_shapes=[
            pltpu.VMEM((1, heads), F32),
            pltpu.VMEM((1, heads), F32),
            pltpu.VMEM((heads, emb), F32),
        ],
    )(h, p0, p1, *warrs)


def kernel(x, edge_attr, params, edge_index):
    n = x.shape[0]
    e_cnt = edge_attr.shape[0]

    row = edge_index[0]
    col = edge_index[1]
    zeros64 = jnp.zeros((n, 64), F32)
    zeros48 = jnp.zeros((n, 48), F32)

    h = _tc_embed(x, params["node_embed"]["W"],
                  params["node_embed"]["b"].reshape(1, -1))

    num_layers = len(params["layers"])
    e_cur = edge_attr
    for li, lp in enumerate(params["layers"]):
        ws = {
            "wn0h": lp["nm0"]["W"][0:64],
            "wn0e": lp["nm0"]["W"][64:96],
            "bn0": lp["nm0"]["b"].reshape(1, -1),
            "wn1": lp["nm1"]["W"], "bn1": lp["nm1"]["b"].reshape(1, -1),
            "wn2": lp["nm2"]["W"], "bn2": lp["nm2"]["b"].reshape(1, -1),
            "we0h": lp["em0"]["W"][0:64],
            "we0c": lp["em0"]["W"][64:128],
            "we0e": lp["em0"]["W"][128:160],
            "be0": lp["em0"]["b"].reshape(1, -1),
            "we1": lp["em1"]["W"], "be1": lp["em1"]["b"].reshape(1, -1),
        }
        if li == 0:
            ws["wee"] = params["edge_embed"]["W"]
            ws["bee"] = params["edge_embed"]["b"].reshape(1, -1)
        hr, hc = _sc_gather_pair(h, row, col)
        dh, e_cur = _tc_edge_mlp(hr, hc, e_cur, ws,
                                 first=(li == 0), last=(li == num_layers - 1))
        p0, p1 = _sc_scatter_add(dh, row, zeros64)
        h = _tc_add3(h, p0, p1)

    q0, q1 = _sc_scatter_add(e_cur, row, zeros48)

    pw = params["pool"]
    hw = params["head"]
    pool_ws = {
        "wnp": pw["node_proj"]["W"], "bnp": pw["node_proj"]["b"].reshape(1, -1),
        "wep": pw["edge_proj"]["W"], "bep": pw["edge_proj"]["b"].reshape(1, -1),
        "wq": pw["Wq"]["W"], "bq": pw["Wq"]["b"].reshape(1, -1),
        "query": pw["query"],
        "wk": pw["Wk"]["W"], "bk": pw["Wk"]["b"].reshape(1, -1),
        "wv": pw["Wv"]["W"], "bv": pw["Wv"]["b"].reshape(1, -1),
        "wo": pw["Wo"]["W"], "bo": pw["Wo"]["b"].reshape(1, -1),
        "wh0": hw["h0"]["W"], "bh0": hw["h0"]["b"].reshape(1, -1),
        "ln_g": hw["ln_g"].reshape(1, -1), "ln_b": hw["ln_b"].reshape(1, -1),
        "wh1": hw["h1"]["W"], "bh1": hw["h1"]["b"].reshape(1, -1),
    }
    return _tc_pool_head(h, q0, q1, pool_ws)


# speedup vs baseline: 1.0721x; 1.0721x over previous
import functools

import jax
import jax.numpy as jnp
from jax import lax
from jax.experimental import pallas as pl
from jax.experimental.pallas import tpu as pltpu
from jax.experimental.pallas import tpu_sc as plsc

F32 = jnp.float32
CH = 125
KG = 4


def _relu(t):
    return jnp.maximum(t, 0.0)


def _mm(a, b):
    return jnp.dot(a, b, preferred_element_type=F32)


def _sc_mesh():
    return plsc.VectorSubcoreMesh(core_axis_name="c", subcore_axis_name="s")


@functools.partial(jax.jit, static_argnums=())
def _sc_gather_pair(table, idx_row, idx_col):
    n, d = table.shape
    e = idx_row.shape[0]
    info = plsc.get_sparse_core_info()
    nc, ns = info.num_cores, info.num_subcores
    nw = nc * ns
    ch = CH
    pw = e // (nw * ch)
    ng = pw // KG
    grp = KG * ch
    idx_row3 = idx_row.reshape(nw, pw, ch)
    idx_col3 = idx_col.reshape(nw, pw, ch)

    @functools.partial(
        pl.kernel,
        out_type=(jax.ShapeDtypeStruct((e, d), F32),
                  jax.ShapeDtypeStruct((e, d), F32)),
        mesh=_sc_mesh(),
        scratch_types=[
            pltpu.VMEM((pw, ch), jnp.int32),
            pltpu.VMEM((2, grp, d), F32),
            pltpu.SemaphoreType.DMA,
            pltpu.SemaphoreType.DMA((2,)),
        ],
        compiler_params=pltpu.CompilerParams(use_tc_tiling_on_sc=False),
    )
    def k(table_h, ir_h, ic_h, or_h, oc_h, idx_v, buf_v, gsem, osem):
        cid = lax.axis_index("c")
        sid = lax.axis_index("s")
        wid = sid * nc + cid
        rbase = wid * pw * ch

        def run(idx_h, out_h):
            pltpu.sync_copy(idx_h.at[wid], idx_v)

            def body(g, carry):
                par = lax.rem(g, 2)
                @pl.when(g >= 2)
                def _():
                    pltpu.make_async_copy(
                        buf_v.at[par],
                        out_h.at[pl.ds(rbase + g * grp, grp)],
                        osem.at[par]).wait()

                for kk in range(KG):
                    pltpu.async_copy(
                        table_h.at[idx_v.at[g * KG + kk]],
                        buf_v.at[par, pl.ds(kk * ch, ch)], gsem)
                for kk in range(KG):
                    pltpu.make_async_copy(
                        table_h.at[idx_v.at[g * KG + kk]],
                        buf_v.at[par, pl.ds(kk * ch, ch)], gsem).wait()
                pltpu.async_copy(
                    buf_v.at[par],
                    out_h.at[pl.ds(rbase + g * grp, grp)], osem.at[par])
                return carry

            lax.fori_loop(0, ng, body, 0)
            for par in range(2):
                pltpu.make_async_copy(
                    buf_v.at[par],
                    out_h.at[pl.ds(rbase, grp)], osem.at[par]).wait()

        run(ir_h, or_h)
        run(ic_h, oc_h)

    return k(table, idx_row3, idx_col3)


def _sc_scatter_add(vals, idx, zeros):
    e, w = vals.shape
    n = zeros.shape[0]
    info = plsc.get_sparse_core_info()
    nc, ns = info.num_cores, info.num_subcores
    nw = nc * ns
    ch = CH
    pw = e // (nw * ch)
    ng = pw // KG
    grp = KG * ch
    idx3 = idx.reshape(nw, pw, ch)
    rows_per = (n // ns) // 8 * 8
    rows_last = n - rows_per * (ns - 1)

    @functools.partial(
        pl.kernel,
        out_type=(jax.ShapeDtypeStruct((n, w), F32),
                  jax.ShapeDtypeStruct((n, w), F32)),
        mesh=_sc_mesh(),
        scratch_types=[
            pltpu.VMEM((pw, ch), jnp.int32),
            pltpu.VMEM((2, grp, w), F32),
            pltpu.VMEM_SHARED((n, w), F32),
            pltpu.SemaphoreType.DMA((2,)),
            pltpu.SemaphoreType.DMA((2,)),
        ],
        compiler_params=pltpu.CompilerParams(use_tc_tiling_on_sc=False),
    )
    def k(vals_h, idx_h, zeros_h, p0_h, p1_h, idx_v, buf_v, acc, lsem, ssem):
        cid = lax.axis_index("c")
        sid = lax.axis_index("s")
        wid = sid * nc + cid
        rbase = wid * pw * ch

        pltpu.sync_copy(idx_h.at[wid], idx_v)

        def init_and_out(fn):
            @pl.when(sid < ns - 1)
            def _():
                fn(pl.ds(sid * rows_per, rows_per))

            @pl.when(sid == ns - 1)
            def _():
                fn(pl.ds((ns - 1) * rows_per, rows_last))

        init_and_out(lambda sl: pltpu.sync_copy(zeros_h.at[sl], acc.at[sl]))
        plsc.subcore_barrier()

        def load_grp(g, par):
            pltpu.async_copy(vals_h.at[pl.ds(rbase + g * grp, grp)],
                             buf_v.at[par], lsem.at[par])

        def drain_adds(g, par):
            for kk in range(KG):
                pltpu.make_async_copy(
                    buf_v.at[par, pl.ds(kk * ch, ch)],
                    acc.at[idx_v.at[g * KG + kk]], ssem.at[par]).wait()

        load_grp(0, 0)

        def body(g, carry):
            par = lax.rem(g, 2)
            pltpu.make_async_copy(vals_h.at[pl.ds(rbase + g * grp, grp)],
                                  buf_v.at[par], lsem.at[par]).wait()

            @pl.when(g >= 1)
            def _():
                drain_adds(g - 1, 1 - par)

            @pl.when(g + 1 < ng)
            def _():
                load_grp(g + 1, 1 - par)

            for kk in range(KG):
                pltpu.async_copy(
                    buf_v.at[par, pl.ds(kk * ch, ch)],
                    acc.at[idx_v.at[g * KG + kk]], ssem.at[par], add=True)
            return carry

        lax.fori_loop(0, ng, body, 0)
        drain_adds(ng - 1, (ng - 1) % 2)
        plsc.subcore_barrier()

        @pl.when(cid == 0)
        def _():
            init_and_out(lambda sl: pltpu.sync_copy(acc.at[sl], p0_h.at[sl]))

        @pl.when(cid == 1)
        def _():
            init_and_out(lambda sl: pltpu.sync_copy(acc.at[sl], p1_h.at[sl]))

    return k(vals, idx3, zeros)


def _tc_embed(x, w, b):
    n, din = x.shape
    dout = w.shape[1]
    blk = n // 5

    def body(x_ref, w_ref, b_ref, o_ref):
        o_ref[...] = _mm(x_ref[...], w_ref[...]) + b_ref[...]

    return pl.pallas_call(
        body,
        grid=(n // blk,),
        in_specs=[
            pl.BlockSpec((blk, din), lambda i: (i, 0)),
            pl.BlockSpec((din, dout), lambda i: (0, 0)),
            pl.BlockSpec((1, dout), lambda i: (0, 0)),
        ],
        out_specs=pl.BlockSpec((blk, dout), lambda i: (i, 0)),
        out_shape=jax.ShapeDtypeStruct((n, dout), F32),
    )(x, w, b)


def _tc_add3(h, p0, p1):
    n, d = h.shape
    blk = n // 5

    def body(a_ref, b_ref, c_ref, o_ref):
        o_ref[...] = a_ref[...] + b_ref[...] + c_ref[...]

    return pl.pallas_call(
        body,
        grid=(n // blk,),
        in_specs=[pl.BlockSpec((blk, d), lambda i: (i, 0))] * 3,
        out_specs=pl.BlockSpec((blk, d), lambda i: (i, 0)),
        out_shape=jax.ShapeDtypeStruct((n, d), F32),
    )(h, p0, p1)


def _tc_edge_mlp(hr, hc, e_in, ws, first, last):
    e_cnt = hr.shape[0]
    blk = 2000
    grid = e_cnt // blk
    ein_w = e_in.shape[1]
    eout_w = 48 if last else 32

    names = (["wee", "bee"] if first else []) + [
        "w1", "b1", "wn1", "bn1", "w2", "b2",
    ]
    warrs = [ws[nm] for nm in names]

    def body(hr_ref, hc_ref, e_ref, *rest):
        wr = {nm: r[...] for nm, r in zip(names, rest[:len(names)])}
        dh_ref, eo_ref = rest[len(names):]
        if first:
            e_b = _mm(e_ref[...], wr["wee"]) + wr["bee"]
        else:
            e_b = e_ref[...]
        x = jnp.concatenate([hr_ref[...], hc_ref[...], e_b], axis=1)
        y = _relu(_mm(x, wr["w1"]) + wr["b1"])
        d2 = _relu(_mm(y[:, :64], wr["wn1"]) + wr["bn1"])
        z = jnp.concatenate([d2, y[:, 64:128]], axis=1)
        o = _mm(z, wr["w2"]) + wr["b2"]
        dh_ref[...] = o[:, :64]
        e_new = e_b + o[:, 64:96]
        if last:
            eo_ref[...] = jnp.concatenate(
                [e_new, jnp.ones((blk, 1), F32), jnp.zeros((blk, 15), F32)],
                axis=1)
        else:
            eo_ref[...] = e_new

    in_specs = [
        pl.BlockSpec((blk, 64), lambda i: (i, 0)),
        pl.BlockSpec((blk, 64), lambda i: (i, 0)),
        pl.BlockSpec((blk, ein_w), lambda i: (i, 0)),
    ] + [pl.BlockSpec(w.shape, lambda i: (0, 0)) for w in warrs]

    return pl.pallas_call(
        body,
        grid=(grid,),
        in_specs=in_specs,
        out_specs=[
            pl.BlockSpec((blk, 64), lambda i: (i, 0)),
            pl.BlockSpec((blk, eout_w), lambda i: (i, 0)),
        ],
        out_shape=[
            jax.ShapeDtypeStruct((e_cnt, 64), F32),
            jax.ShapeDtypeStruct((e_cnt, eout_w), F32),
        ],
    )(hr, hc, e_in, *warrs)


def _tc_pool_head(h, p0, p1, ws):
    n = h.shape[0]
    blk = n // 5
    grid = n // blk
    names = ["wnp", "bnp", "wep", "bep", "wq", "bq", "query",
             "wk", "bk", "wv", "bv", "wo", "bo",
             "wh0", "bh0", "ln_g", "ln_b", "wh1", "bh1"]
    warrs = [ws[nm] for nm in names]

    heads, dh_ = 4, 64
    emb = heads * dh_

    def body(h_ref, p0_ref, p1_ref, *rest):
        wr = {nm: r[...] for nm, r in zip(names, rest[:len(names)])}
        out_ref = rest[len(names)]
        m_s, d_s, num_s = rest[len(names) + 1:]
        i = pl.program_id(0)

        @pl.when(i == 0)
        def _():
            m_s[...] = jnp.full((1, heads), -1e30, F32)
            d_s[...] = jnp.zeros((1, heads), F32)
            num_s[...] = jnp.zeros((heads, emb), F32)

        seg = p0_ref[...] + p1_ref[...]
        cnt = seg[:, 32:33]
        sums = _mm(seg[:, :32], wr["wep"]) + cnt * wr["bep"]
        hp = (_mm(h_ref[...], wr["wnp"]) + wr["bnp"]
              + sums / jnp.maximum(cnt, 1.0))
        kk = _mm(hp, wr["wk"]) + wr["bk"]
        vv = _mm(hp, wr["wv"]) + wr["bv"]

        q = _mm(wr["query"], wr["wq"]) + wr["bq"]
        colh = lax.broadcasted_iota(jnp.int32, (emb, heads), 0) // dh_
        rowh = lax.broadcasted_iota(jnp.int32, (emb, heads), 1)
        hsel = (colh == rowh).astype(F32)
        s = _mm(kk * q, hsel) * (1.0 / 8.0)

        m_prev = m_s[...]
        bm = jnp.max(s, axis=0, keepdims=True)
        m_new = jnp.maximum(m_prev, bm)
        corr = jnp.exp(m_prev - m_new)
        wgt = jnp.exp(s - m_new)
        d_s[...] = d_s[...] * corr + jnp.sum(wgt, axis=0, keepdims=True)
        num_s[...] = (num_s[...] * jnp.transpose(corr)
                      + lax.dot_general(wgt, vv, (((0,), (0,)), ((), ())),
                                        preferred_element_type=F32))
        m_s[...] = m_new

        @pl.when(i == grid - 1)
        def _():
            bd = jnp.transpose(hsel)
            o = jnp.sum(num_s[...] * bd, axis=0, keepdims=True)
            den = _mm(d_s[...], bd)
            o = o / den
            z = _relu(_mm(o, wr["wo"]) + wr["bo"])
            z = _relu(_mm(z, wr["wh0"]) + wr["bh0"])
            mu = jnp.mean(z, axis=-1, keepdims=True)
            var = jnp.mean((z - mu) ** 2, axis=-1, keepdims=True)
            zn = (z - mu) * lax.rsqrt(var + 1e-5)
            zn = zn * wr["ln_g"] + wr["ln_b"]
            out_ref[...] = _mm(zn, wr["wh1"]) + wr["bh1"]

    in_specs = [
        pl.BlockSpec((blk, 64), lambda i: (i, 0)),
        pl.BlockSpec((blk, 48), lambda i: (i, 0)),
        pl.BlockSpec((blk, 48), lambda i: (i, 0)),
    ] + [pl.BlockSpec(w.shape, lambda i: (0, 0)) for w in warrs]

    return pl.pallas_call(
        body,
        grid=(grid,),
        in_specs=in_specs,
        out_specs=pl.BlockSpec((1, 1024), lambda i: (0, 0)),
        out_shape=jax.ShapeDtypeStruct((1, 1024), F32),
        scratch_shapes=[
            pltpu.VMEM((1, heads), F32),
            pltpu.VMEM((1, heads), F32),
            pltpu.VMEM((heads, emb), F32),
        ],
    )(h, p0, p1, *warrs)


def kernel(x, edge_attr, params, edge_index):
    n = x.shape[0]
    e_cnt = edge_attr.shape[0]

    row = edge_index[0]
    col = edge_index[1]
    zeros64 = jnp.zeros((n, 64), F32)
    zeros48 = jnp.zeros((n, 48), F32)

    h = _tc_embed(x, params["node_embed"]["W"],
                  params["node_embed"]["b"].reshape(1, -1))

    num_layers = len(params["layers"])
    e_cur = edge_attr
    for li, lp in enumerate(params["layers"]):
        wn0, we0 = lp["nm0"]["W"], lp["em0"]["W"]
        w1 = jnp.concatenate([
            jnp.concatenate([wn0[0:64], we0[0:64]], axis=1),
            jnp.concatenate([jnp.zeros((64, 64), F32), we0[64:128]], axis=1),
            jnp.concatenate([wn0[64:96], we0[128:160]], axis=1),
        ], axis=0)
        b1 = jnp.concatenate([lp["nm0"]["b"], lp["em0"]["b"]]).reshape(1, -1)
        w2 = jnp.concatenate([
            jnp.concatenate([lp["nm2"]["W"], jnp.zeros((128, 32), F32)],
                            axis=1),
            jnp.concatenate([jnp.zeros((64, 64), F32), lp["em1"]["W"]],
                            axis=1),
        ], axis=0)
        b2 = jnp.concatenate([lp["nm2"]["b"], lp["em1"]["b"]]).reshape(1, -1)
        ws = {
            "w1": w1, "b1": b1,
            "wn1": lp["nm1"]["W"], "bn1": lp["nm1"]["b"].reshape(1, -1),
            "w2": w2, "b2": b2,
        }
        if li == 0:
            ws["wee"] = params["edge_embed"]["W"]
            ws["bee"] = params["edge_embed"]["b"].reshape(1, -1)
        hr, hc = _sc_gather_pair(h, row, col)
        dh, e_cur = _tc_edge_mlp(hr, hc, e_cur, ws,
                                 first=(li == 0), last=(li == num_layers - 1))
        p0, p1 = _sc_scatter_add(dh, row, zeros64)
        h = _tc_add3(h, p0, p1)

    q0, q1 = _sc_scatter_add(e_cur, row, zeros48)

    pw = params["pool"]
    hw = params["head"]
    pool_ws = {
        "wnp": pw["node_proj"]["W"], "bnp": pw["node_proj"]["b"].reshape(1, -1),
        "wep": pw["edge_proj"]["W"], "bep": pw["edge_proj"]["b"].reshape(1, -1),
        "wq": pw["Wq"]["W"], "bq": pw["Wq"]["b"].reshape(1, -1),
        "query": pw["query"],
        "wk": pw["Wk"]["W"], "bk": pw["Wk"]["b"].reshape(1, -1),
        "wv": pw["Wv"]["W"], "bv": pw["Wv"]["b"].reshape(1, -1),
        "wo": pw["Wo"]["W"], "bo": pw["Wo"]["b"].reshape(1, -1),
        "wh0": hw["h0"]["W"], "bh0": hw["h0"]["b"].reshape(1, -1),
        "ln_g": hw["ln_g"].reshape(1, -1), "ln_b": hw["ln_b"].reshape(1, -1),
        "wh1": hw["h1"]["W"], "bh1": hw["h1"]["b"].reshape(1, -1),
    }
    return _tc_pool_head(h, q0, q1, pool_ws)


# speedup vs baseline: 1.9845x; 1.8510x over previous
import functools

import jax
import jax.numpy as jnp
from jax import lax
from jax.experimental import pallas as pl
from jax.experimental.pallas import tpu as pltpu
from jax.experimental.pallas import tpu_sc as plsc

F32 = jnp.float32
CH = 125
KG = 4


def _relu(t):
    return jnp.maximum(t, 0.0)


def _mm(a, b):
    return jnp.dot(a, b, preferred_element_type=F32)


def _sc_mesh():
    return plsc.VectorSubcoreMesh(core_axis_name="c", subcore_axis_name="s")


@functools.partial(jax.jit, static_argnums=())
def _sc_gather_pair(table, idx_row, idx_col):
    n, d = table.shape
    e = idx_row.shape[0]
    info = plsc.get_sparse_core_info()
    nc, ns = info.num_cores, info.num_subcores
    nw = nc * ns
    ch = CH
    pw = e // (nw * ch)
    ng = pw // KG
    grp = KG * ch
    idx_row3 = idx_row.reshape(nw, pw, ch)
    idx_col3 = idx_col.reshape(nw, pw, ch)

    @functools.partial(
        pl.kernel,
        out_type=jax.ShapeDtypeStruct((e, 2 * d), F32),
        mesh=_sc_mesh(),
        scratch_types=[
            pltpu.VMEM((pw, ch), jnp.int32),
            pltpu.VMEM((2, grp, d), F32),
            pltpu.SemaphoreType.DMA,
            pltpu.SemaphoreType.DMA((2,)),
        ],
        compiler_params=pltpu.CompilerParams(use_tc_tiling_on_sc=False),
    )
    def k(table_h, ir_h, ic_h, out_h, idx_v, buf_v, gsem, osem):
        cid = lax.axis_index("c")
        sid = lax.axis_index("s")
        wid = sid * nc + cid
        rbase = wid * pw * ch

        def run(idx_h, c0):
            pltpu.sync_copy(idx_h.at[wid], idx_v)

            def dst(g):
                return out_h.at[pl.ds(rbase + g * grp, grp), pl.ds(c0, d)]

            def body(g, carry):
                par = lax.rem(g, 2)
                @pl.when(g >= 2)
                def _():
                    pltpu.make_async_copy(buf_v.at[par], dst(g),
                                          osem.at[par]).wait()

                for kk in range(KG):
                    pltpu.async_copy(
                        table_h.at[idx_v.at[g * KG + kk]],
                        buf_v.at[par, pl.ds(kk * ch, ch)], gsem)
                for kk in range(KG):
                    pltpu.make_async_copy(
                        table_h.at[idx_v.at[g * KG + kk]],
                        buf_v.at[par, pl.ds(kk * ch, ch)], gsem).wait()
                pltpu.async_copy(buf_v.at[par], dst(g), osem.at[par])
                return carry

            lax.fori_loop(0, ng, body, 0)
            for par in range(2):
                pltpu.make_async_copy(buf_v.at[par], dst(0),
                                      osem.at[par]).wait()

        run(ir_h, 0)
        run(ic_h, d)

    return k(table, idx_row3, idx_col3)


def _sc_scatter_add(vals, idx, zeros, c0):
    e = vals.shape[0]
    w = zeros.shape[1]
    n = zeros.shape[0]
    info = plsc.get_sparse_core_info()
    nc, ns = info.num_cores, info.num_subcores
    nw = nc * ns
    ch = CH
    pw = e // (nw * ch)
    ng = pw // KG
    grp = KG * ch
    idx3 = idx.reshape(nw, pw, ch)
    rows_per = (n // ns) // 8 * 8
    rows_last = n - rows_per * (ns - 1)

    @functools.partial(
        pl.kernel,
        out_type=(jax.ShapeDtypeStruct((n, w), F32),
                  jax.ShapeDtypeStruct((n, w), F32)),
        mesh=_sc_mesh(),
        scratch_types=[
            pltpu.VMEM((pw, ch), jnp.int32),
            pltpu.VMEM((2, grp, w), F32),
            pltpu.VMEM_SHARED((n, w), F32),
            pltpu.SemaphoreType.DMA((2,)),
            pltpu.SemaphoreType.DMA((2,)),
        ],
        compiler_params=pltpu.CompilerParams(use_tc_tiling_on_sc=False),
    )
    def k(vals_h, idx_h, zeros_h, p0_h, p1_h, idx_v, buf_v, acc, lsem, ssem):
        cid = lax.axis_index("c")
        sid = lax.axis_index("s")
        wid = sid * nc + cid
        rbase = wid * pw * ch

        pltpu.sync_copy(idx_h.at[wid], idx_v)

        def init_and_out(fn):
            @pl.when(sid < ns - 1)
            def _():
                fn(pl.ds(sid * rows_per, rows_per))

            @pl.when(sid == ns - 1)
            def _():
                fn(pl.ds((ns - 1) * rows_per, rows_last))

        init_and_out(lambda sl: pltpu.sync_copy(zeros_h.at[sl], acc.at[sl]))
        plsc.subcore_barrier()

        def load_grp(g, par):
            pltpu.async_copy(
                vals_h.at[pl.ds(rbase + g * grp, grp), pl.ds(c0, w)],
                buf_v.at[par], lsem.at[par])

        def drain_adds(g, par):
            for kk in range(KG):
                pltpu.make_async_copy(
                    buf_v.at[par, pl.ds(kk * ch, ch)],
                    acc.at[idx_v.at[g * KG + kk]], ssem.at[par]).wait()

        load_grp(0, 0)

        def body(g, carry):
            par = lax.rem(g, 2)
            pltpu.make_async_copy(
                vals_h.at[pl.ds(rbase + g * grp, grp), pl.ds(c0, w)],
                buf_v.at[par], lsem.at[par]).wait()

            @pl.when(g >= 1)
            def _():
                drain_adds(g - 1, 1 - par)

            @pl.when(g + 1 < ng)
            def _():
                load_grp(g + 1, 1 - par)

            for kk in range(KG):
                pltpu.async_copy(
                    buf_v.at[par, pl.ds(kk * ch, ch)],
                    acc.at[idx_v.at[g * KG + kk]], ssem.at[par], add=True)
            return carry

        lax.fori_loop(0, ng, body, 0)
        drain_adds(ng - 1, (ng - 1) % 2)
        plsc.subcore_barrier()

        @pl.when(cid == 0)
        def _():
            init_and_out(lambda sl: pltpu.sync_copy(acc.at[sl], p0_h.at[sl]))

        @pl.when(cid == 1)
        def _():
            init_and_out(lambda sl: pltpu.sync_copy(acc.at[sl], p1_h.at[sl]))

    return k(vals, idx3, zeros)


def _tc_embed(x, w, b):
    n, din = x.shape
    dout = w.shape[1]
    blk = n // 5

    def body(x_ref, w_ref, b_ref, o_ref):
        o_ref[...] = _mm(x_ref[...], w_ref[...]) + b_ref[...]

    return pl.pallas_call(
        body,
        grid=(n // blk,),
        in_specs=[
            pl.BlockSpec((blk, din), lambda i: (i, 0)),
            pl.BlockSpec((din, dout), lambda i: (0, 0)),
            pl.BlockSpec((1, dout), lambda i: (0, 0)),
        ],
        out_specs=pl.BlockSpec((blk, dout), lambda i: (i, 0)),
        out_shape=jax.ShapeDtypeStruct((n, dout), F32),
    )(x, w, b)


def _tc_add3(h, p0, p1):
    n, d = h.shape
    blk = n // 5

    def body(a_ref, b_ref, c_ref, o_ref):
        o_ref[...] = a_ref[...] + b_ref[...] + c_ref[...]

    return pl.pallas_call(
        body,
        grid=(n // blk,),
        in_specs=[pl.BlockSpec((blk, d), lambda i: (i, 0))] * 3,
        out_specs=pl.BlockSpec((blk, d), lambda i: (i, 0)),
        out_shape=jax.ShapeDtypeStruct((n, d), F32),
    )(h, p0, p1)


def _tc_edge_mlp(ghh, e_in, ws, first, last):
    e_cnt = ghh.shape[0]
    blk = 2000
    grid = e_cnt // blk
    ein_w = e_in.shape[1]

    names = (["wee", "bee"] if first else []) + [
        "w1", "b1", "wn1", "bn1", "w2", "b2",
    ]
    warrs = [ws[nm] for nm in names]

    def body(g_ref, e_ref, *rest):
        wr = {nm: r[...] for nm, r in zip(names, rest[:len(names)])}
        out_ref = rest[len(names)]
        if first:
            e_b = _mm(e_ref[...], wr["wee"]) + wr["bee"]
        else:
            e_b = e_ref[:, 64:96]
        x = jnp.concatenate([g_ref[...], e_b], axis=1)
        y = _relu(_mm(x, wr["w1"]) + wr["b1"])
        d2 = _relu(_mm(y[:, :64], wr["wn1"]) + wr["bn1"])
        z = jnp.concatenate([d2, y[:, 64:128]], axis=1)
        o = _mm(z, wr["w2"]) + wr["b2"]
        e_new = e_b + o[:, 64:96]
        cnt = jnp.full((blk, 1), 1.0 if last else 0.0, F32)
        out_ref[...] = jnp.concatenate(
            [o[:, :64], e_new, cnt, jnp.zeros((blk, 31), F32)], axis=1)

    in_specs = [
        pl.BlockSpec((blk, 128), lambda i: (i, 0)),
        pl.BlockSpec((blk, ein_w), lambda i: (i, 0)),
    ] + [pl.BlockSpec(w.shape, lambda i: (0, 0)) for w in warrs]

    return pl.pallas_call(
        body,
        grid=(grid,),
        in_specs=in_specs,
        out_specs=pl.BlockSpec((blk, 128), lambda i: (i, 0)),
        out_shape=jax.ShapeDtypeStruct((e_cnt, 128), F32),
    )(ghh, e_in, *warrs)


def _tc_pool_head(h, p0, p1, ws):
    n = h.shape[0]
    blk = n // 5
    grid = n // blk
    names = ["wnp", "bnp", "wep", "bep", "wq", "bq", "query",
             "wk", "bk", "wv", "bv", "wo", "bo",
             "wh0", "bh0", "ln_g", "ln_b", "wh1", "bh1"]
    warrs = [ws[nm] for nm in names]

    heads, dh_ = 4, 64
    emb = heads * dh_

    def body(h_ref, p0_ref, p1_ref, *rest):
        wr = {nm: r[...] for nm, r in zip(names, rest[:len(names)])}
        out_ref = rest[len(names)]
        m_s, d_s, num_s = rest[len(names) + 1:]
        i = pl.program_id(0)

        @pl.when(i == 0)
        def _():
            m_s[...] = jnp.full((1, heads), -1e30, F32)
            d_s[...] = jnp.zeros((1, heads), F32)
            num_s[...] = jnp.zeros((heads, emb), F32)

        seg = p0_ref[...] + p1_ref[...]
        cnt = seg[:, 32:33]
        sums = _mm(seg[:, :32], wr["wep"]) + cnt * wr["bep"]
        hp = (_mm(h_ref[...], wr["wnp"]) + wr["bnp"]
              + sums / jnp.maximum(cnt, 1.0))
        kk = _mm(hp, wr["wk"]) + wr["bk"]
        vv = _mm(hp, wr["wv"]) + wr["bv"]

        q = _mm(wr["query"], wr["wq"]) + wr["bq"]
        colh = lax.broadcasted_iota(jnp.int32, (emb, heads), 0) // dh_
        rowh = lax.broadcasted_iota(jnp.int32, (emb, heads), 1)
        hsel = (colh == rowh).astype(F32)
        s = _mm(kk * q, hsel) * (1.0 / 8.0)

        m_prev = m_s[...]
        bm = jnp.max(s, axis=0, keepdims=True)
        m_new = jnp.maximum(m_prev, bm)
        corr = jnp.exp(m_prev - m_new)
        wgt = jnp.exp(s - m_new)
        d_s[...] = d_s[...] * corr + jnp.sum(wgt, axis=0, keepdims=True)
        num_s[...] = (num_s[...] * jnp.transpose(corr)
                      + lax.dot_general(wgt, vv, (((0,), (0,)), ((), ())),
                                        preferred_element_type=F32))
        m_s[...] = m_new

        @pl.when(i == grid - 1)
        def _():
            bd = jnp.transpose(hsel)
            o = jnp.sum(num_s[...] * bd, axis=0, keepdims=True)
            den = _mm(d_s[...], bd)
            o = o / den
            z = _relu(_mm(o, wr["wo"]) + wr["bo"])
            z = _relu(_mm(z, wr["wh0"]) + wr["bh0"])
            mu = jnp.mean(z, axis=-1, keepdims=True)
            var = jnp.mean((z - mu) ** 2, axis=-1, keepdims=True)
            zn = (z - mu) * lax.rsqrt(var + 1e-5)
            zn = zn * wr["ln_g"] + wr["ln_b"]
            out_ref[...] = _mm(zn, wr["wh1"]) + wr["bh1"]

    in_specs = [
        pl.BlockSpec((blk, 64), lambda i: (i, 0)),
        pl.BlockSpec((blk, 48), lambda i: (i, 0)),
        pl.BlockSpec((blk, 48), lambda i: (i, 0)),
    ] + [pl.BlockSpec(w.shape, lambda i: (0, 0)) for w in warrs]

    return pl.pallas_call(
        body,
        grid=(grid,),
        in_specs=in_specs,
        out_specs=pl.BlockSpec((1, 1024), lambda i: (0, 0)),
        out_shape=jax.ShapeDtypeStruct((1, 1024), F32),
        scratch_shapes=[
            pltpu.VMEM((1, heads), F32),
            pltpu.VMEM((1, heads), F32),
            pltpu.VMEM((heads, emb), F32),
        ],
    )(h, p0, p1, *warrs)


def kernel(x, edge_attr, params, edge_index):
    n = x.shape[0]
    e_cnt = edge_attr.shape[0]

    row = edge_index[0]
    col = edge_index[1]
    zeros64 = jnp.zeros((n, 64), F32)
    zeros48 = jnp.zeros((n, 48), F32)

    h = _tc_embed(x, params["node_embed"]["W"],
                  params["node_embed"]["b"].reshape(1, -1))

    num_layers = len(params["layers"])
    e_cur = edge_attr
    for li, lp in enumerate(params["layers"]):
        wn0, we0 = lp["nm0"]["W"], lp["em0"]["W"]
        w1 = jnp.concatenate([
            jnp.concatenate([wn0[0:64], we0[0:64]], axis=1),
            jnp.concatenate([jnp.zeros((64, 64), F32), we0[64:128]], axis=1),
            jnp.concatenate([wn0[64:96], we0[128:160]], axis=1),
        ], axis=0)
        b1 = jnp.concatenate([lp["nm0"]["b"], lp["em0"]["b"]]).reshape(1, -1)
        w2 = jnp.concatenate([
            jnp.concatenate([lp["nm2"]["W"], jnp.zeros((128, 32), F32)],
                            axis=1),
            jnp.concatenate([jnp.zeros((64, 64), F32), lp["em1"]["W"]],
                            axis=1),
        ], axis=0)
        b2 = jnp.concatenate([lp["nm2"]["b"], lp["em1"]["b"]]).reshape(1, -1)
        ws = {
            "w1": w1, "b1": b1,
            "wn1": lp["nm1"]["W"], "bn1": lp["nm1"]["b"].reshape(1, -1),
            "w2": w2, "b2": b2,
        }
        if li == 0:
            ws["wee"] = params["edge_embed"]["W"]
            ws["bee"] = params["edge_embed"]["b"].reshape(1, -1)
        ghh = _sc_gather_pair(h, row, col)
        e_cur = _tc_edge_mlp(ghh, e_cur, ws,
                             first=(li == 0), last=(li == num_layers - 1))
        p0, p1 = _sc_scatter_add(e_cur, row, zeros64, 0)
        h = _tc_add3(h, p0, p1)

    q0, q1 = _sc_scatter_add(e_cur, row, zeros48, 64)

    pw = params["pool"]
    hw = params["head"]
    pool_ws = {
        "wnp": pw["node_proj"]["W"], "bnp": pw["node_proj"]["b"].reshape(1, -1),
        "wep": pw["edge_proj"]["W"], "bep": pw["edge_proj"]["b"].reshape(1, -1),
        "wq": pw["Wq"]["W"], "bq": pw["Wq"]["b"].reshape(1, -1),
        "query": pw["query"],
        "wk": pw["Wk"]["W"], "bk": pw["Wk"]["b"].reshape(1, -1),
        "wv": pw["Wv"]["W"], "bv": pw["Wv"]["b"].reshape(1, -1),
        "wo": pw["Wo"]["W"], "bo": pw["Wo"]["b"].reshape(1, -1),
        "wh0": hw["h0"]["W"], "bh0": hw["h0"]["b"].reshape(1, -1),
        "ln_g": hw["ln_g"].reshape(1, -1), "ln_b": hw["ln_b"].reshape(1, -1),
        "wh1": hw["h1"]["W"], "bh1": hw["h1"]["b"].reshape(1, -1),
    }
    return _tc_pool_head(h, q0, q1, pool_ws)


# speedup vs baseline: 2.1058x; 1.0611x over previous
import functools

import jax
import jax.numpy as jnp
from jax import lax
from jax.experimental import pallas as pl
from jax.experimental.pallas import tpu as pltpu
from jax.experimental.pallas import tpu_sc as plsc

F32 = jnp.float32
CH = 125
KG = 4


def _relu(t):
    return jnp.maximum(t, 0.0)


def _mm(a, b):
    return jnp.dot(a, b, preferred_element_type=F32)


def _sc_mesh():
    return plsc.VectorSubcoreMesh(core_axis_name="c", subcore_axis_name="s")


@functools.partial(jax.jit, static_argnums=())
def _sc_gather_pair(table, idx_row, idx_col):
    n, d = table.shape
    e = idx_row.shape[0]
    info = plsc.get_sparse_core_info()
    nc, ns = info.num_cores, info.num_subcores
    nw = nc * ns
    ch = CH
    pw = e // (nw * ch)
    ng = pw // KG
    grp = KG * ch
    idx_row3 = idx_row.reshape(nw, pw, ch)
    idx_col3 = idx_col.reshape(nw, pw, ch)

    @functools.partial(
        pl.kernel,
        out_type=jax.ShapeDtypeStruct((e, 2 * d), F32),
        mesh=_sc_mesh(),
        scratch_types=[
            pltpu.VMEM((pw, ch), jnp.int32),
            pltpu.VMEM((2, grp, d), F32),
            pltpu.SemaphoreType.DMA,
            pltpu.SemaphoreType.DMA((2,)),
        ],
        compiler_params=pltpu.CompilerParams(use_tc_tiling_on_sc=False),
    )
    def k(table_h, ir_h, ic_h, out_h, idx_v, buf_v, gsem, osem):
        cid = lax.axis_index("c")
        sid = lax.axis_index("s")
        wid = sid * nc + cid
        rbase = wid * pw * ch

        def run(idx_h, c0):
            pltpu.sync_copy(idx_h.at[wid], idx_v)

            def dst(g):
                return out_h.at[pl.ds(rbase + g * grp, grp), pl.ds(c0, d)]

            def body(g, carry):
                par = lax.rem(g, 2)
                @pl.when(g >= 2)
                def _():
                    pltpu.make_async_copy(buf_v.at[par], dst(g),
                                          osem.at[par]).wait()

                for kk in range(KG):
                    pltpu.async_copy(
                        table_h.at[idx_v.at[g * KG + kk]],
                        buf_v.at[par, pl.ds(kk * ch, ch)], gsem)
                for kk in range(KG):
                    pltpu.make_async_copy(
                        table_h.at[idx_v.at[g * KG + kk]],
                        buf_v.at[par, pl.ds(kk * ch, ch)], gsem).wait()
                pltpu.async_copy(buf_v.at[par], dst(g), osem.at[par])
                return carry

            lax.fori_loop(0, ng, body, 0)
            for par in range(2):
                pltpu.make_async_copy(buf_v.at[par], dst(0),
                                      osem.at[par]).wait()

        run(ir_h, 0)
        run(ic_h, d)

    return k(table, idx_row3, idx_col3)


def _sc_scatter_add(vals, idx, zeros, c0):
    e = vals.shape[0]
    w = zeros.shape[1]
    n = zeros.shape[0]
    info = plsc.get_sparse_core_info()
    nc, ns = info.num_cores, info.num_subcores
    nw = nc * ns
    ch = CH
    pw = e // (nw * ch)
    ng = pw // KG
    grp = KG * ch
    idx3 = idx.reshape(nw, pw, ch)
    rows_per = (n // ns) // 8 * 8
    rows_last = n - rows_per * (ns - 1)

    @functools.partial(
        pl.kernel,
        out_type=(jax.ShapeDtypeStruct((n, w), F32),
                  jax.ShapeDtypeStruct((n, w), F32)),
        mesh=_sc_mesh(),
        scratch_types=[
            pltpu.VMEM((pw, ch), jnp.int32),
            pltpu.VMEM((2, grp, w), F32),
            pltpu.VMEM_SHARED((n, w), F32),
            pltpu.SemaphoreType.DMA((2,)),
            pltpu.SemaphoreType.DMA((2,)),
        ],
        compiler_params=pltpu.CompilerParams(use_tc_tiling_on_sc=False),
    )
    def k(vals_h, idx_h, zeros_h, p0_h, p1_h, idx_v, buf_v, acc, lsem, ssem):
        cid = lax.axis_index("c")
        sid = lax.axis_index("s")
        wid = sid * nc + cid
        rbase = wid * pw * ch

        pltpu.sync_copy(idx_h.at[wid], idx_v)

        def init_and_out(fn):
            @pl.when(sid < ns - 1)
            def _():
                fn(pl.ds(sid * rows_per, rows_per))

            @pl.when(sid == ns - 1)
            def _():
                fn(pl.ds((ns - 1) * rows_per, rows_last))

        init_and_out(lambda sl: pltpu.sync_copy(zeros_h.at[sl], acc.at[sl]))
        plsc.subcore_barrier()

        def load_grp(g, par):
            pltpu.async_copy(
                vals_h.at[pl.ds(rbase + g * grp, grp), pl.ds(c0, w)],
                buf_v.at[par], lsem.at[par])

        def drain_adds(g, par):
            for kk in range(KG):
                pltpu.make_async_copy(
                    buf_v.at[par, pl.ds(kk * ch, ch)],
                    acc.at[idx_v.at[g * KG + kk]], ssem.at[par]).wait()

        load_grp(0, 0)

        def body(g, carry):
            par = lax.rem(g, 2)
            pltpu.make_async_copy(
                vals_h.at[pl.ds(rbase + g * grp, grp), pl.ds(c0, w)],
                buf_v.at[par], lsem.at[par]).wait()

            @pl.when(g >= 1)
            def _():
                drain_adds(g - 1, 1 - par)

            @pl.when(g + 1 < ng)
            def _():
                load_grp(g + 1, 1 - par)

            for kk in range(KG):
                pltpu.async_copy(
                    buf_v.at[par, pl.ds(kk * ch, ch)],
                    acc.at[idx_v.at[g * KG + kk]], ssem.at[par], add=True)
            return carry

        lax.fori_loop(0, ng, body, 0)
        drain_adds(ng - 1, (ng - 1) % 2)
        plsc.subcore_barrier()

        @pl.when(cid == 0)
        def _():
            init_and_out(lambda sl: pltpu.sync_copy(acc.at[sl], p0_h.at[sl]))

        @pl.when(cid == 1)
        def _():
            init_and_out(lambda sl: pltpu.sync_copy(acc.at[sl], p1_h.at[sl]))

    return k(vals, idx3, zeros)


def _tc_embed(x, w, b):
    n, din = x.shape
    dout = w.shape[1]
    blk = n // 5

    def body(x_ref, w_ref, b_ref, o_ref):
        o_ref[...] = _mm(x_ref[...], w_ref[...]) + b_ref[...]

    return pl.pallas_call(
        body,
        grid=(n // blk,),
        in_specs=[
            pl.BlockSpec((blk, din), lambda i: (i, 0)),
            pl.BlockSpec((din, dout), lambda i: (0, 0)),
            pl.BlockSpec((1, dout), lambda i: (0, 0)),
        ],
        out_specs=pl.BlockSpec((blk, dout), lambda i: (i, 0)),
        out_shape=jax.ShapeDtypeStruct((n, dout), F32),
    )(x, w, b)


def _tc_addn(*arrs):
    n, d = arrs[0].shape
    blk = n // 5

    def body(*refs):
        refs[-1][...] = sum(r[...] for r in refs[:-1])

    return pl.pallas_call(
        body,
        grid=(n // blk,),
        in_specs=[pl.BlockSpec((blk, d), lambda i: (i, 0))] * len(arrs),
        out_specs=pl.BlockSpec((blk, d), lambda i: (i, 0)),
        out_shape=jax.ShapeDtypeStruct((n, d), F32),
    )(*arrs)


def _tc_edge_mlp(ghh, e_in, ws, first, last):
    e_cnt = ghh.shape[0]
    blk = 2000
    grid = e_cnt // blk
    ein_w = e_in.shape[1]

    names = (["wee", "bee"] if first else []) + [
        "w1", "b1", "wn1", "bn1", "w2", "b2",
    ]
    warrs = [ws[nm] for nm in names]

    def body(g_ref, e_ref, *rest):
        wr = {nm: r[...] for nm, r in zip(names, rest[:len(names)])}
        out_ref = rest[len(names)]
        if first:
            e_b = _mm(e_ref[...], wr["wee"]) + wr["bee"]
        else:
            e_b = e_ref[:, 64:96]
        x = jnp.concatenate([g_ref[...], e_b], axis=1)
        y = _relu(_mm(x, wr["w1"]) + wr["b1"])
        d2 = _relu(_mm(y[:, :64], wr["wn1"]) + wr["bn1"])
        z = jnp.concatenate([d2, y[:, 64:128]], axis=1)
        o = _mm(z, wr["w2"]) + wr["b2"]
        e_new = e_b + o[:, 64:96]
        cnt = jnp.full((blk, 1), 1.0 if last else 0.0, F32)
        out_ref[...] = jnp.concatenate(
            [o[:, :64], e_new, cnt, jnp.zeros((blk, 31), F32)], axis=1)

    in_specs = [
        pl.BlockSpec((blk, 128), lambda i: (i, 0)),
        pl.BlockSpec((blk, ein_w), lambda i: (i, 0)),
    ] + [pl.BlockSpec(w.shape, lambda i: (0, 0)) for w in warrs]

    return pl.pallas_call(
        body,
        grid=(grid,),
        in_specs=in_specs,
        out_specs=pl.BlockSpec((blk, 128), lambda i: (i, 0)),
        out_shape=jax.ShapeDtypeStruct((e_cnt, 128), F32),
    )(ghh, e_in, *warrs)


def _tc_pool_head(h, qs, ws):
    n = h.shape[0]
    blk = n // 5
    grid = n // blk
    names = ["wnp", "bnp", "wep", "bep", "wq", "bq", "query",
             "wk", "bk", "wv", "bv", "wo", "bo",
             "wh0", "bh0", "ln_g", "ln_b", "wh1", "bh1"]
    warrs = [ws[nm] for nm in names]

    heads, dh_ = 4, 64
    emb = heads * dh_

    nq = len(qs)

    def body(h_ref, *rest):
        q_refs = rest[:nq]
        wr = {nm: r[...] for nm, r in zip(names, rest[nq:nq + len(names)])}
        out_ref = rest[nq + len(names)]
        m_s, d_s, num_s = rest[nq + len(names) + 1:]
        i = pl.program_id(0)

        @pl.when(i == 0)
        def _():
            m_s[...] = jnp.full((1, heads), -1e30, F32)
            d_s[...] = jnp.zeros((1, heads), F32)
            num_s[...] = jnp.zeros((heads, emb), F32)

        seg = sum(q[...] for q in q_refs)
        cnt = seg[:, 32:33]
        sums = _mm(seg[:, :32], wr["wep"]) + cnt * wr["bep"]
        hp = (_mm(h_ref[...], wr["wnp"]) + wr["bnp"]
              + sums / jnp.maximum(cnt, 1.0))
        kk = _mm(hp, wr["wk"]) + wr["bk"]
        vv = _mm(hp, wr["wv"]) + wr["bv"]

        q = _mm(wr["query"], wr["wq"]) + wr["bq"]
        colh = lax.broadcasted_iota(jnp.int32, (emb, heads), 0) // dh_
        rowh = lax.broadcasted_iota(jnp.int32, (emb, heads), 1)
        hsel = (colh == rowh).astype(F32)
        s = _mm(kk * q, hsel) * (1.0 / 8.0)

        m_prev = m_s[...]
        bm = jnp.max(s, axis=0, keepdims=True)
        m_new = jnp.maximum(m_prev, bm)
        corr = jnp.exp(m_prev - m_new)
        wgt = jnp.exp(s - m_new)
        d_s[...] = d_s[...] * corr + jnp.sum(wgt, axis=0, keepdims=True)
        num_s[...] = (num_s[...] * jnp.transpose(corr)
                      + lax.dot_general(wgt, vv, (((0,), (0,)), ((), ())),
                                        preferred_element_type=F32))
        m_s[...] = m_new

        @pl.when(i == grid - 1)
        def _():
            bd = jnp.transpose(hsel)
            o = jnp.sum(num_s[...] * bd, axis=0, keepdims=True)
            den = _mm(d_s[...], bd)
            o = o / den
            z = _relu(_mm(o, wr["wo"]) + wr["bo"])
            z = _relu(_mm(z, wr["wh0"]) + wr["bh0"])
            mu = jnp.mean(z, axis=-1, keepdims=True)
            var = jnp.mean((z - mu) ** 2, axis=-1, keepdims=True)
            zn = (z - mu) * lax.rsqrt(var + 1e-5)
            zn = zn * wr["ln_g"] + wr["ln_b"]
            out_ref[...] = _mm(zn, wr["wh1"]) + wr["bh1"]

    in_specs = [
        pl.BlockSpec((blk, 64), lambda i: (i, 0)),
    ] + [pl.BlockSpec((blk, 48), lambda i: (i, 0))] * nq + [
        pl.BlockSpec(w.shape, lambda i: (0, 0)) for w in warrs]

    return pl.pallas_call(
        body,
        grid=(grid,),
        in_specs=in_specs,
        out_specs=pl.BlockSpec((1, 1024), lambda i: (0, 0)),
        out_shape=jax.ShapeDtypeStruct((1, 1024), F32),
        scratch_shapes=[
            pltpu.VMEM((1, heads), F32),
            pltpu.VMEM((1, heads), F32),
            pltpu.VMEM((heads, emb), F32),
        ],
    )(h, *qs, *warrs)


def kernel(x, edge_attr, params, edge_index):
    n = x.shape[0]
    e_cnt = edge_attr.shape[0]

    row = edge_index[0]
    col = edge_index[1]
    zeros64 = jnp.zeros((n, 64), F32)
    zeros48 = jnp.zeros((n, 48), F32)

    h = _tc_embed(x, params["node_embed"]["W"],
                  params["node_embed"]["b"].reshape(1, -1))

    num_layers = len(params["layers"])
    nsplit = 2
    hh = e_cnt // nsplit
    parts = [(row[k * hh:(k + 1) * hh], col[k * hh:(k + 1) * hh])
             for k in range(nsplit)]
    e_curs = [edge_attr[k * hh:(k + 1) * hh] for k in range(nsplit)]
    for li, lp in enumerate(params["layers"]):
        wn0, we0 = lp["nm0"]["W"], lp["em0"]["W"]
        w1 = jnp.concatenate([
            jnp.concatenate([wn0[0:64], we0[0:64]], axis=1),
            jnp.concatenate([jnp.zeros((64, 64), F32), we0[64:128]], axis=1),
            jnp.concatenate([wn0[64:96], we0[128:160]], axis=1),
        ], axis=0)
        b1 = jnp.concatenate([lp["nm0"]["b"], lp["em0"]["b"]]).reshape(1, -1)
        w2 = jnp.concatenate([
            jnp.concatenate([lp["nm2"]["W"], jnp.zeros((128, 32), F32)],
                            axis=1),
            jnp.concatenate([jnp.zeros((64, 64), F32), lp["em1"]["W"]],
                            axis=1),
        ], axis=0)
        b2 = jnp.concatenate([lp["nm2"]["b"], lp["em1"]["b"]]).reshape(1, -1)
        ws = {
            "w1": w1, "b1": b1,
            "wn1": lp["nm1"]["W"], "bn1": lp["nm1"]["b"].reshape(1, -1),
            "w2": w2, "b2": b2,
        }
        if li == 0:
            ws["wee"] = params["edge_embed"]["W"]
            ws["bee"] = params["edge_embed"]["b"].reshape(1, -1)
        gs = [_sc_gather_pair(h, r, c) for (r, c) in parts]
        e_curs = [_tc_edge_mlp(gs[k], e_curs[k], ws, first=(li == 0),
                               last=(li == num_layers - 1))
                  for k in range(nsplit)]
        partials = []
        for k in range(nsplit):
            partials += list(_sc_scatter_add(e_curs[k], parts[k][0],
                                             zeros64, 0))
        h = _tc_addn(h, *partials)

    qs = []
    for k in range(nsplit):
        qs += list(_sc_scatter_add(e_curs[k], parts[k][0], zeros48, 64))

    pw = params["pool"]
    hw = params["head"]
    pool_ws = {
        "wnp": pw["node_proj"]["W"], "bnp": pw["node_proj"]["b"].reshape(1, -1),
        "wep": pw["edge_proj"]["W"], "bep": pw["edge_proj"]["b"].reshape(1, -1),
        "wq": pw["Wq"]["W"], "bq": pw["Wq"]["b"].reshape(1, -1),
        "query": pw["query"],
        "wk": pw["Wk"]["W"], "bk": pw["Wk"]["b"].reshape(1, -1),
        "wv": pw["Wv"]["W"], "bv": pw["Wv"]["b"].reshape(1, -1),
        "wo": pw["Wo"]["W"], "bo": pw["Wo"]["b"].reshape(1, -1),
        "wh0": hw["h0"]["W"], "bh0": hw["h0"]["b"].reshape(1, -1),
        "ln_g": hw["ln_g"].reshape(1, -1), "ln_b": hw["ln_b"].reshape(1, -1),
        "wh1": hw["h1"]["W"], "bh1": hw["h1"]["b"].reshape(1, -1),
    }
    return _tc_pool_head(h, qs, pool_ws)


# speedup vs baseline: 2.1390x; 1.0158x over previous
import functools

import jax
import jax.numpy as jnp
from jax import lax
from jax.experimental import pallas as pl
from jax.experimental.pallas import tpu as pltpu
from jax.experimental.pallas import tpu_sc as plsc

F32 = jnp.float32
CH = 125
KG = 4


def _relu(t):
    return jnp.maximum(t, 0.0)


def _mm(a, b):
    return jnp.dot(a, b, preferred_element_type=F32)


def _sc_mesh():
    return plsc.VectorSubcoreMesh(core_axis_name="c", subcore_axis_name="s")


@functools.partial(jax.jit, static_argnums=())
def _sc_gather_pair(table, idx_row, idx_col):
    n, d = table.shape
    e = idx_row.shape[0]
    info = plsc.get_sparse_core_info()
    nc, ns = info.num_cores, info.num_subcores
    nw = nc * ns
    ch = CH
    pw = e // (nw * ch)
    ng = pw // KG
    grp = KG * ch
    idx_row3 = idx_row.reshape(nw, pw, ch)
    idx_col3 = idx_col.reshape(nw, pw, ch)

    @functools.partial(
        pl.kernel,
        out_type=jax.ShapeDtypeStruct((e, 2 * d), F32),
        mesh=_sc_mesh(),
        scratch_types=[
            pltpu.VMEM((pw, ch), jnp.int32),
            pltpu.VMEM((2, grp, d), F32),
            pltpu.SemaphoreType.DMA,
            pltpu.SemaphoreType.DMA((2,)),
        ],
        compiler_params=pltpu.CompilerParams(use_tc_tiling_on_sc=False),
    )
    def k(table_h, ir_h, ic_h, out_h, idx_v, buf_v, gsem, osem):
        cid = lax.axis_index("c")
        sid = lax.axis_index("s")
        wid = sid * nc + cid
        rbase = wid * pw * ch

        def run(idx_h, c0):
            pltpu.sync_copy(idx_h.at[wid], idx_v)

            def dst(g):
                return out_h.at[pl.ds(rbase + g * grp, grp), pl.ds(c0, d)]

            def body(g, carry):
                par = lax.rem(g, 2)
                @pl.when(g >= 2)
                def _():
                    pltpu.make_async_copy(buf_v.at[par], dst(g),
                                          osem.at[par]).wait()

                for kk in range(KG):
                    pltpu.async_copy(
                        table_h.at[idx_v.at[g * KG + kk]],
                        buf_v.at[par, pl.ds(kk * ch, ch)], gsem)
                for kk in range(KG):
                    pltpu.make_async_copy(
                        table_h.at[idx_v.at[g * KG + kk]],
                        buf_v.at[par, pl.ds(kk * ch, ch)], gsem).wait()
                pltpu.async_copy(buf_v.at[par], dst(g), osem.at[par])
                return carry

            lax.fori_loop(0, ng, body, 0)
            for par in range(2):
                pltpu.make_async_copy(buf_v.at[par], dst(0),
                                      osem.at[par]).wait()

        run(ir_h, 0)
        run(ic_h, d)

    return k(table, idx_row3, idx_col3)


def _sc_scatter_add(vals, idx, zeros, c0):
    e = vals.shape[0]
    w = zeros.shape[1]
    n = zeros.shape[0]
    info = plsc.get_sparse_core_info()
    nc, ns = info.num_cores, info.num_subcores
    nw = nc * ns
    ch = CH
    pw = e // (nw * ch)
    ng = pw // KG
    grp = KG * ch
    idx3 = idx.reshape(nw, pw, ch)
    rows_per = (n // ns) // 8 * 8
    rows_last = n - rows_per * (ns - 1)

    @functools.partial(
        pl.kernel,
        out_type=(jax.ShapeDtypeStruct((n, w), F32),
                  jax.ShapeDtypeStruct((n, w), F32)),
        mesh=_sc_mesh(),
        scratch_types=[
            pltpu.VMEM((pw, ch), jnp.int32),
            pltpu.VMEM((2, grp, w), F32),
            pltpu.VMEM_SHARED((n, w), F32),
            pltpu.SemaphoreType.DMA((2,)),
            pltpu.SemaphoreType.DMA((2,)),
        ],
        compiler_params=pltpu.CompilerParams(use_tc_tiling_on_sc=False),
    )
    def k(vals_h, idx_h, zeros_h, p0_h, p1_h, idx_v, buf_v, acc, lsem, ssem):
        cid = lax.axis_index("c")
        sid = lax.axis_index("s")
        wid = sid * nc + cid
        rbase = wid * pw * ch

        pltpu.sync_copy(idx_h.at[wid], idx_v)

        def init_and_out(fn):
            @pl.when(sid < ns - 1)
            def _():
                fn(pl.ds(sid * rows_per, rows_per))

            @pl.when(sid == ns - 1)
            def _():
                fn(pl.ds((ns - 1) * rows_per, rows_last))

        init_and_out(lambda sl: pltpu.sync_copy(zeros_h.at[sl], acc.at[sl]))
        plsc.subcore_barrier()

        def load_grp(g, par):
            pltpu.async_copy(
                vals_h.at[pl.ds(rbase + g * grp, grp), pl.ds(c0, w)],
                buf_v.at[par], lsem.at[par])

        def drain_adds(g, par):
            for kk in range(KG):
                pltpu.make_async_copy(
                    buf_v.at[par, pl.ds(kk * ch, ch)],
                    acc.at[idx_v.at[g * KG + kk]], ssem.at[par]).wait()

        load_grp(0, 0)

        def body(g, carry):
            par = lax.rem(g, 2)
            pltpu.make_async_copy(
                vals_h.at[pl.ds(rbase + g * grp, grp), pl.ds(c0, w)],
                buf_v.at[par], lsem.at[par]).wait()

            @pl.when(g >= 1)
            def _():
                drain_adds(g - 1, 1 - par)

            @pl.when(g + 1 < ng)
            def _():
                load_grp(g + 1, 1 - par)

            for kk in range(KG):
                pltpu.async_copy(
                    buf_v.at[par, pl.ds(kk * ch, ch)],
                    acc.at[idx_v.at[g * KG + kk]], ssem.at[par], add=True)
            return carry

        lax.fori_loop(0, ng, body, 0)
        drain_adds(ng - 1, (ng - 1) % 2)
        plsc.subcore_barrier()

        @pl.when(cid == 0)
        def _():
            init_and_out(lambda sl: pltpu.sync_copy(acc.at[sl], p0_h.at[sl]))

        @pl.when(cid == 1)
        def _():
            init_and_out(lambda sl: pltpu.sync_copy(acc.at[sl], p1_h.at[sl]))

    return k(vals, idx3, zeros)


def _tc_embed(x, w, b):
    n, din = x.shape
    dout = w.shape[1]
    blk = n // 5

    def body(x_ref, w_ref, b_ref, o_ref):
        o_ref[...] = _mm(x_ref[...], w_ref[...]) + b_ref[...]

    return pl.pallas_call(
        body,
        grid=(n // blk,),
        in_specs=[
            pl.BlockSpec((blk, din), lambda i: (i, 0)),
            pl.BlockSpec((din, dout), lambda i: (0, 0)),
            pl.BlockSpec((1, dout), lambda i: (0, 0)),
        ],
        out_specs=pl.BlockSpec((blk, dout), lambda i: (i, 0)),
        out_shape=jax.ShapeDtypeStruct((n, dout), F32),
    )(x, w, b)


def _tc_addn(*arrs):
    n, d = arrs[0].shape
    blk = n // 5

    def body(*refs):
        refs[-1][...] = sum(r[...] for r in refs[:-1])

    return pl.pallas_call(
        body,
        grid=(n // blk,),
        in_specs=[pl.BlockSpec((blk, d), lambda i: (i, 0))] * len(arrs),
        out_specs=pl.BlockSpec((blk, d), lambda i: (i, 0)),
        out_shape=jax.ShapeDtypeStruct((n, d), F32),
    )(*arrs)


def _tc_edge_mlp(ghh, e_in, ws, first, last, eoff=0):
    e_cnt = ghh.shape[0]
    blk = 2000
    grid = e_cnt // blk
    ein_w = e_in.shape[1]

    names = (["wee", "bee"] if first else []) + [
        "w1h", "w1e", "b1", "wn1", "bn1", "w2d", "w2u", "b2",
    ]
    warrs = [ws[nm] for nm in names]

    def body(g_ref, e_ref, *rest):
        wr = {nm: r[...] for nm, r in zip(names, rest[:len(names)])}
        out_ref = rest[len(names)]
        if first:
            e_b = _mm(e_ref[...], wr["wee"]) + wr["bee"]
        else:
            e_b = e_ref[:, 64:96]
        y = _relu(_mm(g_ref[...], wr["w1h"]) + _mm(e_b, wr["w1e"])
                  + wr["b1"])
        d2 = _relu(_mm(y[:, :64], wr["wn1"]) + wr["bn1"])
        o = _mm(d2, wr["w2d"]) + _mm(y[:, 64:128], wr["w2u"]) + wr["b2"]
        e_new = e_b + o[:, 64:96]
        cnt = jnp.full((blk, 1), 1.0 if last else 0.0, F32)
        out_ref[...] = jnp.concatenate(
            [o[:, :64], e_new, cnt, jnp.zeros((blk, 31), F32)], axis=1)

    eblk = eoff // blk
    in_specs = [
        pl.BlockSpec((blk, 128), lambda i: (i, 0)),
        pl.BlockSpec((blk, ein_w), lambda i: (i + eblk, 0)),
    ] + [pl.BlockSpec(w.shape, lambda i: (0, 0)) for w in warrs]

    return pl.pallas_call(
        body,
        grid=(grid,),
        in_specs=in_specs,
        out_specs=pl.BlockSpec((blk, 128), lambda i: (i, 0)),
        out_shape=jax.ShapeDtypeStruct((e_cnt, 128), F32),
    )(ghh, e_in, *warrs)


def _tc_pool_head(h, qs, ws):
    n = h.shape[0]
    blk = n // 5
    grid = n // blk
    names = ["wnp", "bnp", "wep", "bep", "wq", "bq", "query",
             "wk", "bk", "wv", "bv", "wo", "bo",
             "wh0", "bh0", "ln_g", "ln_b", "wh1", "bh1"]
    warrs = [ws[nm] for nm in names]

    heads, dh_ = 4, 64
    emb = heads * dh_

    nq = len(qs)

    def body(h_ref, *rest):
        q_refs = rest[:nq]
        wr = {nm: r[...] for nm, r in zip(names, rest[nq:nq + len(names)])}
        out_ref = rest[nq + len(names)]
        m_s, d_s, num_s = rest[nq + len(names) + 1:]
        i = pl.program_id(0)

        @pl.when(i == 0)
        def _():
            m_s[...] = jnp.full((1, heads), -1e30, F32)
            d_s[...] = jnp.zeros((1, heads), F32)
            num_s[...] = jnp.zeros((heads, emb), F32)

        seg = sum(q[...] for q in q_refs)
        cnt = seg[:, 32:33]
        sums = _mm(seg[:, :32], wr["wep"]) + cnt * wr["bep"]
        hp = (_mm(h_ref[...], wr["wnp"]) + wr["bnp"]
              + sums / jnp.maximum(cnt, 1.0))
        kk = _mm(hp, wr["wk"]) + wr["bk"]
        vv = _mm(hp, wr["wv"]) + wr["bv"]

        q = _mm(wr["query"], wr["wq"]) + wr["bq"]
        colh = lax.broadcasted_iota(jnp.int32, (emb, heads), 0) // dh_
        rowh = lax.broadcasted_iota(jnp.int32, (emb, heads), 1)
        hsel = (colh == rowh).astype(F32)
        s = _mm(kk * q, hsel) * (1.0 / 8.0)

        m_prev = m_s[...]
        bm = jnp.max(s, axis=0, keepdims=True)
        m_new = jnp.maximum(m_prev, bm)
        corr = jnp.exp(m_prev - m_new)
        wgt = jnp.exp(s - m_new)
        d_s[...] = d_s[...] * corr + jnp.sum(wgt, axis=0, keepdims=True)
        num_s[...] = (num_s[...] * jnp.transpose(corr)
                      + lax.dot_general(wgt, vv, (((0,), (0,)), ((), ())),
                                        preferred_element_type=F32))
        m_s[...] = m_new

        @pl.when(i == grid - 1)
        def _():
            bd = jnp.transpose(hsel)
            o = jnp.sum(num_s[...] * bd, axis=0, keepdims=True)
            den = _mm(d_s[...], bd)
            o = o / den
            z = _relu(_mm(o, wr["wo"]) + wr["bo"])
            z = _relu(_mm(z, wr["wh0"]) + wr["bh0"])
            mu = jnp.mean(z, axis=-1, keepdims=True)
            var = jnp.mean((z - mu) ** 2, axis=-1, keepdims=True)
            zn = (z - mu) * lax.rsqrt(var + 1e-5)
            zn = zn * wr["ln_g"] + wr["ln_b"]
            out_ref[...] = _mm(zn, wr["wh1"]) + wr["bh1"]

    in_specs = [
        pl.BlockSpec((blk, 64), lambda i: (i, 0)),
    ] + [pl.BlockSpec((blk, 48), lambda i: (i, 0))] * nq + [
        pl.BlockSpec(w.shape, lambda i: (0, 0)) for w in warrs]

    return pl.pallas_call(
        body,
        grid=(grid,),
        in_specs=in_specs,
        out_specs=pl.BlockSpec((1, 1024), lambda i: (0, 0)),
        out_shape=jax.ShapeDtypeStruct((1, 1024), F32),
        scratch_shapes=[
            pltpu.VMEM((1, heads), F32),
            pltpu.VMEM((1, heads), F32),
            pltpu.VMEM((heads, emb), F32),
        ],
    )(h, *qs, *warrs)


def kernel(x, edge_attr, params, edge_index):
    n = x.shape[0]
    e_cnt = edge_attr.shape[0]

    row = edge_index[0]
    col = edge_index[1]
    zeros64 = jnp.zeros((n, 64), F32)
    zeros48 = jnp.zeros((n, 48), F32)

    h = _tc_embed(x, params["node_embed"]["W"],
                  params["node_embed"]["b"].reshape(1, -1))

    num_layers = len(params["layers"])
    nsplit = 2
    hh = e_cnt // nsplit
    parts = [(row[k * hh:(k + 1) * hh], col[k * hh:(k + 1) * hh])
             for k in range(nsplit)]
    e_curs = [edge_attr] * nsplit
    for li, lp in enumerate(params["layers"]):
        wn0, we0 = lp["nm0"]["W"], lp["em0"]["W"]
        w1h = jnp.concatenate([
            jnp.concatenate([wn0[0:64], we0[0:64]], axis=1),
            jnp.concatenate([jnp.zeros((64, 64), F32), we0[64:128]], axis=1),
        ], axis=0)
        w1e = jnp.concatenate([wn0[64:96], we0[128:160]], axis=1)
        b1 = jnp.concatenate([lp["nm0"]["b"], lp["em0"]["b"]]).reshape(1, -1)
        w2d = jnp.concatenate([lp["nm2"]["W"], jnp.zeros((128, 32), F32)],
                              axis=1)
        w2u = jnp.concatenate([jnp.zeros((64, 64), F32), lp["em1"]["W"]],
                              axis=1)
        b2 = jnp.concatenate([lp["nm2"]["b"], lp["em1"]["b"]]).reshape(1, -1)
        ws = {
            "w1h": w1h, "w1e": w1e, "b1": b1,
            "wn1": lp["nm1"]["W"], "bn1": lp["nm1"]["b"].reshape(1, -1),
            "w2d": w2d, "w2u": w2u, "b2": b2,
        }
        if li == 0:
            ws["wee"] = params["edge_embed"]["W"]
            ws["bee"] = params["edge_embed"]["b"].reshape(1, -1)
        gs = [_sc_gather_pair(h, r, c) for (r, c) in parts]
        e_curs = [_tc_edge_mlp(gs[k], e_curs[k], ws, first=(li == 0),
                               last=(li == num_layers - 1),
                               eoff=(k * hh if li == 0 else 0))
                  for k in range(nsplit)]
        partials = []
        for k in range(nsplit):
            partials += list(_sc_scatter_add(e_curs[k], parts[k][0],
                                             zeros64, 0))
        h = _tc_addn(h, *partials)

    qs = []
    for k in range(nsplit):
        qs += list(_sc_scatter_add(e_curs[k], parts[k][0], zeros48, 64))

    pw = params["pool"]
    hw = params["head"]
    pool_ws = {
        "wnp": pw["node_proj"]["W"], "bnp": pw["node_proj"]["b"].reshape(1, -1),
        "wep": pw["edge_proj"]["W"], "bep": pw["edge_proj"]["b"].reshape(1, -1),
        "wq": pw["Wq"]["W"], "bq": pw["Wq"]["b"].reshape(1, -1),
        "query": pw["query"],
        "wk": pw["Wk"]["W"], "bk": pw["Wk"]["b"].reshape(1, -1),
        "wv": pw["Wv"]["W"], "bv": pw["Wv"]["b"].reshape(1, -1),
        "wo": pw["Wo"]["W"], "bo": pw["Wo"]["b"].reshape(1, -1),
        "wh0": hw["h0"]["W"], "bh0": hw["h0"]["b"].reshape(1, -1),
        "ln_g": hw["ln_g"].reshape(1, -1), "ln_b": hw["ln_b"].reshape(1, -1),
        "wh1": hw["h1"]["W"], "bh1": hw["h1"]["b"].reshape(1, -1),
    }
    return _tc_pool_head(h, qs, pool_ws)


# speedup vs baseline: 2.1542x; 1.0071x over previous
import functools

import jax
import jax.numpy as jnp
from jax import lax
from jax.experimental import pallas as pl
from jax.experimental.pallas import tpu as pltpu
from jax.experimental.pallas import tpu_sc as plsc

F32 = jnp.float32
CH = 125
KG = 4


def _relu(t):
    return jnp.maximum(t, 0.0)


def _mm(a, b):
    return jnp.dot(a, b, preferred_element_type=F32)


def _sc_mesh():
    return plsc.VectorSubcoreMesh(core_axis_name="c", subcore_axis_name="s")


@functools.partial(jax.jit, static_argnums=())
def _sc_gather_pair(table, idx_row, idx_col):
    n, d = table.shape
    e = idx_row.shape[0]
    info = plsc.get_sparse_core_info()
    nc, ns = info.num_cores, info.num_subcores
    nw = nc * ns
    ch = CH
    pw = e // (nw * ch)
    ng = pw // KG
    grp = KG * ch
    idx_row3 = idx_row.reshape(nw, pw, ch)
    idx_col3 = idx_col.reshape(nw, pw, ch)

    @functools.partial(
        pl.kernel,
        out_type=jax.ShapeDtypeStruct((e, 2 * d), F32),
        mesh=_sc_mesh(),
        scratch_types=[
            pltpu.VMEM((pw, ch), jnp.int32),
            pltpu.VMEM((2, grp, d), F32),
            pltpu.SemaphoreType.DMA,
            pltpu.SemaphoreType.DMA((2,)),
        ],
        compiler_params=pltpu.CompilerParams(use_tc_tiling_on_sc=False),
    )
    def k(table_h, ir_h, ic_h, out_h, idx_v, buf_v, gsem, osem):
        cid = lax.axis_index("c")
        sid = lax.axis_index("s")
        wid = sid * nc + cid
        rbase = wid * pw * ch

        def run(idx_h, c0):
            pltpu.sync_copy(idx_h.at[wid], idx_v)

            def dst(g):
                return out_h.at[pl.ds(rbase + g * grp, grp), pl.ds(c0, d)]

            def body(g, carry):
                par = lax.rem(g, 2)
                @pl.when(g >= 2)
                def _():
                    pltpu.make_async_copy(buf_v.at[par], dst(g),
                                          osem.at[par]).wait()

                for kk in range(KG):
                    pltpu.async_copy(
                        table_h.at[idx_v.at[g * KG + kk]],
                        buf_v.at[par, pl.ds(kk * ch, ch)], gsem)
                for kk in range(KG):
                    pltpu.make_async_copy(
                        table_h.at[idx_v.at[g * KG + kk]],
                        buf_v.at[par, pl.ds(kk * ch, ch)], gsem).wait()
                pltpu.async_copy(buf_v.at[par], dst(g), osem.at[par])
                return carry

            lax.fori_loop(0, ng, body, 0)
            for par in range(2):
                pltpu.make_async_copy(buf_v.at[par], dst(0),
                                      osem.at[par]).wait()

        run(ir_h, 0)
        run(ic_h, d)

    return k(table, idx_row3, idx_col3)


def _sc_scatter_add(vals, idx, zeros, c0):
    e = vals.shape[0]
    w = zeros.shape[1]
    n = zeros.shape[0]
    info = plsc.get_sparse_core_info()
    nc, ns = info.num_cores, info.num_subcores
    nw = nc * ns
    ch = CH
    pw = e // (nw * ch)
    ng = pw // KG
    grp = KG * ch
    idx3 = idx.reshape(nw, pw, ch)
    rows_per = (n // ns) // 8 * 8
    rows_last = n - rows_per * (ns - 1)

    @functools.partial(
        pl.kernel,
        out_type=(jax.ShapeDtypeStruct((n, w), F32),
                  jax.ShapeDtypeStruct((n, w), F32)),
        mesh=_sc_mesh(),
        scratch_types=[
            pltpu.VMEM((pw, ch), jnp.int32),
            pltpu.VMEM((2, grp, w), F32),
            pltpu.VMEM_SHARED((n, w), F32),
            pltpu.SemaphoreType.DMA((2,)),
            pltpu.SemaphoreType.DMA((2,)),
        ],
        compiler_params=pltpu.CompilerParams(use_tc_tiling_on_sc=False),
    )
    def k(vals_h, idx_h, zeros_h, p0_h, p1_h, idx_v, buf_v, acc, lsem, ssem):
        cid = lax.axis_index("c")
        sid = lax.axis_index("s")
        wid = sid * nc + cid
        rbase = wid * pw * ch

        pltpu.sync_copy(idx_h.at[wid], idx_v)

        def init_and_out(fn):
            @pl.when(sid < ns - 1)
            def _():
                fn(pl.ds(sid * rows_per, rows_per))

            @pl.when(sid == ns - 1)
            def _():
                fn(pl.ds((ns - 1) * rows_per, rows_last))

        init_and_out(lambda sl: pltpu.sync_copy(zeros_h.at[sl], acc.at[sl]))
        plsc.subcore_barrier()

        def load_grp(g, par):
            pltpu.async_copy(
                vals_h.at[pl.ds(rbase + g * grp, grp), pl.ds(c0, w)],
                buf_v.at[par], lsem.at[par])

        def drain_adds(g, par):
            for kk in range(KG):
                pltpu.make_async_copy(
                    buf_v.at[par, pl.ds(kk * ch, ch)],
                    acc.at[idx_v.at[g * KG + kk]], ssem.at[par]).wait()

        load_grp(0, 0)

        def body(g, carry):
            par = lax.rem(g, 2)
            pltpu.make_async_copy(
                vals_h.at[pl.ds(rbase + g * grp, grp), pl.ds(c0, w)],
                buf_v.at[par], lsem.at[par]).wait()

            @pl.when(g >= 1)
            def _():
                drain_adds(g - 1, 1 - par)

            @pl.when(g + 1 < ng)
            def _():
                load_grp(g + 1, 1 - par)

            for kk in range(KG):
                pltpu.async_copy(
                    buf_v.at[par, pl.ds(kk * ch, ch)],
                    acc.at[idx_v.at[g * KG + kk]], ssem.at[par], add=True)
            return carry

        lax.fori_loop(0, ng, body, 0)
        drain_adds(ng - 1, (ng - 1) % 2)
        plsc.subcore_barrier()

        @pl.when(cid == 0)
        def _():
            init_and_out(lambda sl: pltpu.sync_copy(acc.at[sl], p0_h.at[sl]))

        @pl.when(cid == 1)
        def _():
            init_and_out(lambda sl: pltpu.sync_copy(acc.at[sl], p1_h.at[sl]))

    return k(vals, idx3, zeros)


def _tc_embed(x, w, b):
    n, din = x.shape
    dout = w.shape[1]
    blk = n // 5

    def body(x_ref, w_ref, b_ref, o_ref):
        o_ref[...] = _mm(x_ref[...], w_ref[...]) + b_ref[...]

    return pl.pallas_call(
        body,
        grid=(n // blk,),
        in_specs=[
            pl.BlockSpec((blk, din), lambda i: (i, 0)),
            pl.BlockSpec((din, dout), lambda i: (0, 0)),
            pl.BlockSpec((1, dout), lambda i: (0, 0)),
        ],
        out_specs=pl.BlockSpec((blk, dout), lambda i: (i, 0)),
        out_shape=jax.ShapeDtypeStruct((n, dout), F32),
    )(x, w, b)


def _tc_addn(*arrs):
    n, d = arrs[0].shape
    blk = n // 5

    def body(*refs):
        refs[-1][...] = sum(r[...] for r in refs[:-1])

    return pl.pallas_call(
        body,
        grid=(n // blk,),
        in_specs=[pl.BlockSpec((blk, d), lambda i: (i, 0))] * len(arrs),
        out_specs=pl.BlockSpec((blk, d), lambda i: (i, 0)),
        out_shape=jax.ShapeDtypeStruct((n, d), F32),
    )(*arrs)


def _tc_edge_mlp(ghh, e_in, ws, first, last, eoff=0):
    e_cnt = ghh.shape[0]
    blk = 2000
    grid = e_cnt // blk
    ein_w = e_in.shape[1]

    names = (["wee", "bee"] if first else []) + [
        "w1h", "w1e", "b1", "wn1", "bn1", "w2d", "w2u", "b2",
    ]
    warrs = [ws[nm] for nm in names]

    def body(g_ref, e_ref, *rest):
        wr = {nm: r[...] for nm, r in zip(names, rest[:len(names)])}
        out_ref = rest[len(names)]
        if first:
            e_b = _mm(e_ref[...], wr["wee"]) + wr["bee"]
        else:
            e_b = e_ref[:, 64:96]
        y = _relu(_mm(g_ref[...], wr["w1h"]) + _mm(e_b, wr["w1e"])
                  + wr["b1"])
        d2 = _relu(_mm(y[:, :64], wr["wn1"]) + wr["bn1"])
        o = _mm(d2, wr["w2d"]) + _mm(y[:, 64:128], wr["w2u"]) + wr["b2"]
        e_new = e_b + o[:, 64:96]
        cnt = jnp.full((blk, 1), 1.0 if last else 0.0, F32)
        out_ref[...] = jnp.concatenate(
            [o[:, :64], e_new, cnt, jnp.zeros((blk, 31), F32)], axis=1)

    eblk = eoff // blk
    in_specs = [
        pl.BlockSpec((blk, 128), lambda i: (i, 0)),
        pl.BlockSpec((blk, ein_w), lambda i: (i + eblk, 0)),
    ] + [pl.BlockSpec(w.shape, lambda i: (0, 0)) for w in warrs]

    return pl.pallas_call(
        body,
        grid=(grid,),
        in_specs=in_specs,
        out_specs=pl.BlockSpec((blk, 128), lambda i: (i, 0)),
        out_shape=jax.ShapeDtypeStruct((e_cnt, 128), F32),
    )(ghh, e_in, *warrs)


def _tc_pool_head(h, qs, ws):
    n = h.shape[0]
    blk = n // 5
    grid = n // blk
    names = ["wnp", "bnp", "wep", "bep", "wq", "bq", "query",
             "wk", "bk", "wv", "bv", "wo", "bo",
             "wh0", "bh0", "ln_g", "ln_b", "wh1", "bh1"]
    warrs = [ws[nm] for nm in names]

    heads, dh_ = 4, 64
    emb = heads * dh_

    nq = len(qs)

    def body(h_ref, *rest):
        q_refs = rest[:nq]
        wr = {nm: r[...] for nm, r in zip(names, rest[nq:nq + len(names)])}
        out_ref = rest[nq + len(names)]
        m_s, d_s, num_s = rest[nq + len(names) + 1:]
        i = pl.program_id(0)

        @pl.when(i == 0)
        def _():
            m_s[...] = jnp.full((1, heads), -1e30, F32)
            d_s[...] = jnp.zeros((1, heads), F32)
            num_s[...] = jnp.zeros((heads, emb), F32)

        seg = sum(q[...] for q in q_refs)
        cnt = seg[:, 32:33]
        sums = _mm(seg[:, :32], wr["wep"]) + cnt * wr["bep"]
        hp = (_mm(h_ref[...], wr["wnp"]) + wr["bnp"]
              + sums / jnp.maximum(cnt, 1.0))
        kk = _mm(hp, wr["wk"]) + wr["bk"]
        vv = _mm(hp, wr["wv"]) + wr["bv"]

        q = _mm(wr["query"], wr["wq"]) + wr["bq"]
        colh = lax.broadcasted_iota(jnp.int32, (emb, heads), 0) // dh_
        rowh = lax.broadcasted_iota(jnp.int32, (emb, heads), 1)
        hsel = (colh == rowh).astype(F32)
        s = _mm(kk * q, hsel) * (1.0 / 8.0)

        m_prev = m_s[...]
        bm = jnp.max(s, axis=0, keepdims=True)
        m_new = jnp.maximum(m_prev, bm)
        corr = jnp.exp(m_prev - m_new)
        wgt = jnp.exp(s - m_new)
        d_s[...] = d_s[...] * corr + jnp.sum(wgt, axis=0, keepdims=True)
        num_s[...] = (num_s[...] * jnp.transpose(corr)
                      + lax.dot_general(wgt, vv, (((0,), (0,)), ((), ())),
                                        preferred_element_type=F32))
        m_s[...] = m_new

        @pl.when(i == grid - 1)
        def _():
            bd = jnp.transpose(hsel)
            o = jnp.sum(num_s[...] * bd, axis=0, keepdims=True)
            den = _mm(d_s[...], bd)
            o = o / den
            z = _relu(_mm(o, wr["wo"]) + wr["bo"])
            z = _relu(_mm(z, wr["wh0"]) + wr["bh0"])
            mu = jnp.mean(z, axis=-1, keepdims=True)
            var = jnp.mean((z - mu) ** 2, axis=-1, keepdims=True)
            zn = (z - mu) * lax.rsqrt(var + 1e-5)
            zn = zn * wr["ln_g"] + wr["ln_b"]
            out_ref[...] = _mm(zn, wr["wh1"]) + wr["bh1"]

    in_specs = [
        pl.BlockSpec((blk, 64), lambda i: (i, 0)),
    ] + [pl.BlockSpec((blk, 48), lambda i: (i, 0))] * nq + [
        pl.BlockSpec(w.shape, lambda i: (0, 0)) for w in warrs]

    return pl.pallas_call(
        body,
        grid=(grid,),
        in_specs=in_specs,
        out_specs=pl.BlockSpec((1, 1024), lambda i: (0, 0)),
        out_shape=jax.ShapeDtypeStruct((1, 1024), F32),
        scratch_shapes=[
            pltpu.VMEM((1, heads), F32),
            pltpu.VMEM((1, heads), F32),
            pltpu.VMEM((heads, emb), F32),
        ],
    )(h, *qs, *warrs)


def kernel(x, edge_attr, params, edge_index):
    n = x.shape[0]
    e_cnt = edge_attr.shape[0]

    row = edge_index[0]
    col = edge_index[1]
    zeros64 = jnp.zeros((n, 64), F32)
    zeros48 = jnp.zeros((n, 48), F32)

    h = _tc_embed(x, params["node_embed"]["W"],
                  params["node_embed"]["b"].reshape(1, -1))

    num_layers = len(params["layers"])
    nsplit = 4
    hh = e_cnt // nsplit
    parts = [(row[k * hh:(k + 1) * hh], col[k * hh:(k + 1) * hh])
             for k in range(nsplit)]
    e_curs = [edge_attr] * nsplit
    for li, lp in enumerate(params["layers"]):
        wn0, we0 = lp["nm0"]["W"], lp["em0"]["W"]
        w1h = jnp.concatenate([
            jnp.concatenate([wn0[0:64], we0[0:64]], axis=1),
            jnp.concatenate([jnp.zeros((64, 64), F32), we0[64:128]], axis=1),
        ], axis=0)
        w1e = jnp.concatenate([wn0[64:96], we0[128:160]], axis=1)
        b1 = jnp.concatenate([lp["nm0"]["b"], lp["em0"]["b"]]).reshape(1, -1)
        w2d = jnp.concatenate([lp["nm2"]["W"], jnp.zeros((128, 32), F32)],
                              axis=1)
        w2u = jnp.concatenate([jnp.zeros((64, 64), F32), lp["em1"]["W"]],
                              axis=1)
        b2 = jnp.concatenate([lp["nm2"]["b"], lp["em1"]["b"]]).reshape(1, -1)
        ws = {
            "w1h": w1h, "w1e": w1e, "b1": b1,
            "wn1": lp["nm1"]["W"], "bn1": lp["nm1"]["b"].reshape(1, -1),
            "w2d": w2d, "w2u": w2u, "b2": b2,
        }
        if li == 0:
            ws["wee"] = params["edge_embed"]["W"]
            ws["bee"] = params["edge_embed"]["b"].reshape(1, -1)
        gs = [_sc_gather_pair(h, r, c) for (r, c) in parts]
        e_curs = [_tc_edge_mlp(gs[k], e_curs[k], ws, first=(li == 0),
                               last=(li == num_layers - 1),
                               eoff=(k * hh if li == 0 else 0))
                  for k in range(nsplit)]
        partials = []
        for k in range(nsplit):
            partials += list(_sc_scatter_add(e_curs[k], parts[k][0],
                                             zeros64, 0))
        h = _tc_addn(h, *partials)

    qs = []
    for k in range(nsplit):
        qs += list(_sc_scatter_add(e_curs[k], parts[k][0], zeros48, 64))

    pw = params["pool"]
    hw = params["head"]
    pool_ws = {
        "wnp": pw["node_proj"]["W"], "bnp": pw["node_proj"]["b"].reshape(1, -1),
        "wep": pw["edge_proj"]["W"], "bep": pw["edge_proj"]["b"].reshape(1, -1),
        "wq": pw["Wq"]["W"], "bq": pw["Wq"]["b"].reshape(1, -1),
        "query": pw["query"],
        "wk": pw["Wk"]["W"], "bk": pw["Wk"]["b"].reshape(1, -1),
        "wv": pw["Wv"]["W"], "bv": pw["Wv"]["b"].reshape(1, -1),
        "wo": pw["Wo"]["W"], "bo": pw["Wo"]["b"].reshape(1, -1),
        "wh0": hw["h0"]["W"], "bh0": hw["h0"]["b"].reshape(1, -1),
        "ln_g": hw["ln_g"].reshape(1, -1), "ln_b": hw["ln_b"].reshape(1, -1),
        "wh1": hw["h1"]["W"], "bh1": hw["h1"]["b"].reshape(1, -1),
    }
    return _tc_pool_head(h, qs, pool_ws)


# speedup vs baseline: 2.2158x; 1.0286x over previous
import functools

import jax
import jax.numpy as jnp
from jax import lax
from jax.experimental import pallas as pl
from jax.experimental.pallas import tpu as pltpu
from jax.experimental.pallas import tpu_sc as plsc

F32 = jnp.float32
CH = 125
KG = 4


def _relu(t):
    return jnp.maximum(t, 0.0)


def _mm(a, b):
    return jnp.dot(a, b, preferred_element_type=F32)


def _sc_mesh():
    return plsc.VectorSubcoreMesh(core_axis_name="c", subcore_axis_name="s")


@functools.partial(jax.jit, static_argnums=())
def _sc_gather_pair(table, idx_row, idx_col):
    n, d = table.shape
    e = idx_row.shape[0]
    info = plsc.get_sparse_core_info()
    nc, ns = info.num_cores, info.num_subcores
    nw = nc * ns
    ch = CH
    pw = e // (nw * ch)
    ng = pw // KG
    grp = KG * ch
    idx_row3 = idx_row.reshape(nw, pw, ch)
    idx_col3 = idx_col.reshape(nw, pw, ch)

    @functools.partial(
        pl.kernel,
        out_type=jax.ShapeDtypeStruct((e, 2 * d), F32),
        mesh=_sc_mesh(),
        scratch_types=[
            pltpu.VMEM((pw, ch), jnp.int32),
            pltpu.VMEM((2, grp, d), F32),
            pltpu.SemaphoreType.DMA,
            pltpu.SemaphoreType.DMA((2,)),
        ],
        compiler_params=pltpu.CompilerParams(use_tc_tiling_on_sc=False),
    )
    def k(table_h, ir_h, ic_h, out_h, idx_v, buf_v, gsem, osem):
        cid = lax.axis_index("c")
        sid = lax.axis_index("s")
        wid = sid * nc + cid
        rbase = wid * pw * ch

        def run(idx_h, c0):
            pltpu.sync_copy(idx_h.at[wid], idx_v)

            def dst(g):
                return out_h.at[pl.ds(rbase + g * grp, grp), pl.ds(c0, d)]

            def body(g, carry):
                par = lax.rem(g, 2)
                @pl.when(g >= 2)
                def _():
                    pltpu.make_async_copy(buf_v.at[par], dst(g),
                                          osem.at[par]).wait()

                for kk in range(KG):
                    pltpu.async_copy(
                        table_h.at[idx_v.at[g * KG + kk]],
                        buf_v.at[par, pl.ds(kk * ch, ch)], gsem)
                for kk in range(KG):
                    pltpu.make_async_copy(
                        table_h.at[idx_v.at[g * KG + kk]],
                        buf_v.at[par, pl.ds(kk * ch, ch)], gsem).wait()
                pltpu.async_copy(buf_v.at[par], dst(g), osem.at[par])
                return carry

            lax.fori_loop(0, ng, body, 0)
            for par in range(2):
                pltpu.make_async_copy(buf_v.at[par], dst(0),
                                      osem.at[par]).wait()

        run(ir_h, 0)
        run(ic_h, d)

    return k(table, idx_row3, idx_col3)


def _sc_scatter_add(vals, idx, zeros, c0):
    e = vals.shape[0]
    w = zeros.shape[1]
    n = zeros.shape[0]
    info = plsc.get_sparse_core_info()
    nc, ns = info.num_cores, info.num_subcores
    nw = nc * ns
    ch = CH
    pw = e // (nw * ch)
    ng = pw // KG
    grp = KG * ch
    idx3 = idx.reshape(nw, pw, ch)
    rows_per = (n // ns) // 8 * 8
    rows_last = n - rows_per * (ns - 1)

    @functools.partial(
        pl.kernel,
        out_type=jax.ShapeDtypeStruct((n, 128), F32),
        mesh=_sc_mesh(),
        scratch_types=[
            pltpu.VMEM((pw, ch), jnp.int32),
            pltpu.VMEM((2, grp, w), F32),
            pltpu.VMEM_SHARED((n, w), F32),
            pltpu.SemaphoreType.DMA((2,)),
            pltpu.SemaphoreType.DMA((2,)),
        ],
        compiler_params=pltpu.CompilerParams(use_tc_tiling_on_sc=False),
    )
    def k(vals_h, idx_h, zeros_h, p_h, idx_v, buf_v, acc, lsem, ssem):
        cid = lax.axis_index("c")
        sid = lax.axis_index("s")
        wid = sid * nc + cid
        rbase = wid * pw * ch

        pltpu.sync_copy(idx_h.at[wid], idx_v)

        def init_and_out(fn):
            @pl.when(sid < ns - 1)
            def _():
                fn(pl.ds(sid * rows_per, rows_per))

            @pl.when(sid == ns - 1)
            def _():
                fn(pl.ds((ns - 1) * rows_per, rows_last))

        init_and_out(lambda sl: pltpu.sync_copy(zeros_h.at[sl], acc.at[sl]))
        plsc.subcore_barrier()

        def load_grp(g, par):
            pltpu.async_copy(
                vals_h.at[pl.ds(rbase + g * grp, grp), pl.ds(c0, w)],
                buf_v.at[par], lsem.at[par])

        def drain_adds(g, par):
            for kk in range(KG):
                pltpu.make_async_copy(
                    buf_v.at[par, pl.ds(kk * ch, ch)],
                    acc.at[idx_v.at[g * KG + kk]], ssem.at[par]).wait()

        load_grp(0, 0)

        def body(g, carry):
            par = lax.rem(g, 2)
            pltpu.make_async_copy(
                vals_h.at[pl.ds(rbase + g * grp, grp), pl.ds(c0, w)],
                buf_v.at[par], lsem.at[par]).wait()

            @pl.when(g >= 1)
            def _():
                drain_adds(g - 1, 1 - par)

            @pl.when(g + 1 < ng)
            def _():
                load_grp(g + 1, 1 - par)

            for kk in range(KG):
                pltpu.async_copy(
                    buf_v.at[par, pl.ds(kk * ch, ch)],
                    acc.at[idx_v.at[g * KG + kk]], ssem.at[par], add=True)
            return carry

        lax.fori_loop(0, ng, body, 0)
        drain_adds(ng - 1, (ng - 1) % 2)
        plsc.subcore_barrier()

        init_and_out(lambda sl: pltpu.sync_copy(
            acc.at[sl], p_h.at[sl, pl.ds(cid * 64, w)]))

    return k(vals, idx3, zeros)


def _tc_embed(x, w, b):
    n, din = x.shape
    dout = w.shape[1]
    blk = n // 5

    def body(x_ref, w_ref, b_ref, o_ref):
        o_ref[...] = _mm(x_ref[...], w_ref[...]) + b_ref[...]

    return pl.pallas_call(
        body,
        grid=(n // blk,),
        in_specs=[
            pl.BlockSpec((blk, din), lambda i: (i, 0)),
            pl.BlockSpec((din, dout), lambda i: (0, 0)),
            pl.BlockSpec((1, dout), lambda i: (0, 0)),
        ],
        out_specs=pl.BlockSpec((blk, dout), lambda i: (i, 0)),
        out_shape=jax.ShapeDtypeStruct((n, dout), F32),
    )(x, w, b)


def _tc_addn(h, parts):
    n, d = h.shape
    blk = n // 5

    def body(*refs):
        h_b = refs[0][...]
        acc = h_b
        for r in refs[1:-1]:
            p = r[...]
            acc = acc + p[:, :d] + p[:, d:2 * d]
        refs[-1][...] = acc

    return pl.pallas_call(
        body,
        grid=(n // blk,),
        in_specs=[pl.BlockSpec((blk, d), lambda i: (i, 0))]
        + [pl.BlockSpec((blk, 128), lambda i: (i, 0))] * len(parts),
        out_specs=pl.BlockSpec((blk, d), lambda i: (i, 0)),
        out_shape=jax.ShapeDtypeStruct((n, d), F32),
    )(h, *parts)


def _tc_edge_mlp(ghh, e_in, ws, first, last, eoff=0):
    e_cnt = ghh.shape[0]
    blk = 2000
    grid = e_cnt // blk
    ein_w = e_in.shape[1]

    names = (["wee", "bee"] if first else []) + [
        "w1h", "w1e", "b1", "wn1", "bn1", "w2d", "w2u", "b2",
    ]
    warrs = [ws[nm] for nm in names]

    def body(g_ref, e_ref, *rest):
        wr = {nm: r[...] for nm, r in zip(names, rest[:len(names)])}
        out_ref = rest[len(names)]
        if first:
            e_b = _mm(e_ref[...], wr["wee"]) + wr["bee"]
        else:
            e_b = e_ref[:, 64:96]
        y = _relu(_mm(g_ref[...], wr["w1h"]) + _mm(e_b, wr["w1e"])
                  + wr["b1"])
        d2 = _relu(_mm(y[:, :64], wr["wn1"]) + wr["bn1"])
        o = _mm(d2, wr["w2d"]) + _mm(y[:, 64:128], wr["w2u"]) + wr["b2"]
        e_new = e_b + o[:, 64:96]
        cnt = jnp.full((blk, 1), 1.0 if last else 0.0, F32)
        out_ref[...] = jnp.concatenate(
            [o[:, :64], e_new, cnt, jnp.zeros((blk, 31), F32)], axis=1)

    eblk = eoff // blk
    in_specs = [
        pl.BlockSpec((blk, 128), lambda i: (i, 0)),
        pl.BlockSpec((blk, ein_w), lambda i: (i + eblk, 0)),
    ] + [pl.BlockSpec(w.shape, lambda i: (0, 0)) for w in warrs]

    return pl.pallas_call(
        body,
        grid=(grid,),
        in_specs=in_specs,
        out_specs=pl.BlockSpec((blk, 128), lambda i: (i, 0)),
        out_shape=jax.ShapeDtypeStruct((e_cnt, 128), F32),
    )(ghh, e_in, *warrs)


def _tc_pool_head(h, qs, ws):
    n = h.shape[0]
    blk = n // 5
    grid = n // blk
    names = ["wnp", "bnp", "wep", "bep", "wq", "bq", "query",
             "wk", "bk", "wv", "bv", "wo", "bo",
             "wh0", "bh0", "ln_g", "ln_b", "wh1", "bh1"]
    warrs = [ws[nm] for nm in names]

    heads, dh_ = 4, 64
    emb = heads * dh_

    nq = len(qs)

    def body(h_ref, *rest):
        q_refs = rest[:nq]
        wr = {nm: r[...] for nm, r in zip(names, rest[nq:nq + len(names)])}
        out_ref = rest[nq + len(names)]
        m_s, d_s, num_s = rest[nq + len(names) + 1:]
        i = pl.program_id(0)

        @pl.when(i == 0)
        def _():
            m_s[...] = jnp.full((1, heads), -1e30, F32)
            d_s[...] = jnp.zeros((1, heads), F32)
            num_s[...] = jnp.zeros((heads, emb), F32)

        seg = sum(q[...][:, :48] + q[...][:, 64:112] for q in q_refs)
        cnt = seg[:, 32:33]
        sums = _mm(seg[:, :32], wr["wep"]) + cnt * wr["bep"]
        hp = (_mm(h_ref[...], wr["wnp"]) + wr["bnp"]
              + sums / jnp.maximum(cnt, 1.0))
        kk = _mm(hp, wr["wk"]) + wr["bk"]
        vv = _mm(hp, wr["wv"]) + wr["bv"]

        q = _mm(wr["query"], wr["wq"]) + wr["bq"]
        colh = lax.broadcasted_iota(jnp.int32, (emb, heads), 0) // dh_
        rowh = lax.broadcasted_iota(jnp.int32, (emb, heads), 1)
        hsel = (colh == rowh).astype(F32)
        s = _mm(kk * q, hsel) * (1.0 / 8.0)

        m_prev = m_s[...]
        bm = jnp.max(s, axis=0, keepdims=True)
        m_new = jnp.maximum(m_prev, bm)
        corr = jnp.exp(m_prev - m_new)
        wgt = jnp.exp(s - m_new)
        d_s[...] = d_s[...] * corr + jnp.sum(wgt, axis=0, keepdims=True)
        num_s[...] = (num_s[...] * jnp.transpose(corr)
                      + lax.dot_general(wgt, vv, (((0,), (0,)), ((), ())),
                                        preferred_element_type=F32))
        m_s[...] = m_new

        @pl.when(i == grid - 1)
        def _():
            bd = jnp.transpose(hsel)
            o = jnp.sum(num_s[...] * bd, axis=0, keepdims=True)
            den = _mm(d_s[...], bd)
            o = o / den
            z = _relu(_mm(o, wr["wo"]) + wr["bo"])
            z = _relu(_mm(z, wr["wh0"]) + wr["bh0"])
            mu = jnp.mean(z, axis=-1, keepdims=True)
            var = jnp.mean((z - mu) ** 2, axis=-1, keepdims=True)
            zn = (z - mu) * lax.rsqrt(var + 1e-5)
            zn = zn * wr["ln_g"] + wr["ln_b"]
            out_ref[...] = _mm(zn, wr["wh1"]) + wr["bh1"]

    in_specs = [
        pl.BlockSpec((blk, 64), lambda i: (i, 0)),
    ] + [pl.BlockSpec((blk, 128), lambda i: (i, 0))] * nq + [
        pl.BlockSpec(w.shape, lambda i: (0, 0)) for w in warrs]

    return pl.pallas_call(
        body,
        grid=(grid,),
        in_specs=in_specs,
        out_specs=pl.BlockSpec((1, 1024), lambda i: (0, 0)),
        out_shape=jax.ShapeDtypeStruct((1, 1024), F32),
        scratch_shapes=[
            pltpu.VMEM((1, heads), F32),
            pltpu.VMEM((1, heads), F32),
            pltpu.VMEM((heads, emb), F32),
        ],
    )(h, *qs, *warrs)


def kernel(x, edge_attr, params, edge_index):
    n = x.shape[0]
    e_cnt = edge_attr.shape[0]

    row = edge_index[0]
    col = edge_index[1]
    zeros64 = jnp.zeros((n, 64), F32)
    zeros48 = jnp.zeros((n, 48), F32)

    h = _tc_embed(x, params["node_embed"]["W"],
                  params["node_embed"]["b"].reshape(1, -1))

    num_layers = len(params["layers"])
    nsplit = 4
    hh = e_cnt // nsplit
    parts = [(row[k * hh:(k + 1) * hh], col[k * hh:(k + 1) * hh])
             for k in range(nsplit)]
    e_curs = [edge_attr] * nsplit
    for li, lp in enumerate(params["layers"]):
        wn0, we0 = lp["nm0"]["W"], lp["em0"]["W"]
        w1h = jnp.concatenate([
            jnp.concatenate([wn0[0:64], we0[0:64]], axis=1),
            jnp.concatenate([jnp.zeros((64, 64), F32), we0[64:128]], axis=1),
        ], axis=0)
        w1e = jnp.concatenate([wn0[64:96], we0[128:160]], axis=1)
        b1 = jnp.concatenate([lp["nm0"]["b"], lp["em0"]["b"]]).reshape(1, -1)
        w2d = jnp.concatenate([lp["nm2"]["W"], jnp.zeros((128, 32), F32)],
                              axis=1)
        w2u = jnp.concatenate([jnp.zeros((64, 64), F32), lp["em1"]["W"]],
                              axis=1)
        b2 = jnp.concatenate([lp["nm2"]["b"], lp["em1"]["b"]]).reshape(1, -1)
        ws = {
            "w1h": w1h, "w1e": w1e, "b1": b1,
            "wn1": lp["nm1"]["W"], "bn1": lp["nm1"]["b"].reshape(1, -1),
            "w2d": w2d, "w2u": w2u, "b2": b2,
        }
        if li == 0:
            ws["wee"] = params["edge_embed"]["W"]
            ws["bee"] = params["edge_embed"]["b"].reshape(1, -1)
        gs = [_sc_gather_pair(h, r, c) for (r, c) in parts]
        e_curs = [_tc_edge_mlp(gs[k], e_curs[k], ws, first=(li == 0),
                               last=(li == num_layers - 1),
                               eoff=(k * hh if li == 0 else 0))
                  for k in range(nsplit)]
        partials = [_sc_scatter_add(e_curs[k], parts[k][0], zeros64, 0)
                    for k in range(nsplit)]
        h = _tc_addn(h, partials)

    qs = [_sc_scatter_add(e_curs[k], parts[k][0], zeros48, 64)
          for k in range(nsplit)]

    pw = params["pool"]
    hw = params["head"]
    pool_ws = {
        "wnp": pw["node_proj"]["W"], "bnp": pw["node_proj"]["b"].reshape(1, -1),
        "wep": pw["edge_proj"]["W"], "bep": pw["edge_proj"]["b"].reshape(1, -1),
        "wq": pw["Wq"]["W"], "bq": pw["Wq"]["b"].reshape(1, -1),
        "query": pw["query"],
        "wk": pw["Wk"]["W"], "bk": pw["Wk"]["b"].reshape(1, -1),
        "wv": pw["Wv"]["W"], "bv": pw["Wv"]["b"].reshape(1, -1),
        "wo": pw["Wo"]["W"], "bo": pw["Wo"]["b"].reshape(1, -1),
        "wh0": hw["h0"]["W"], "bh0": hw["h0"]["b"].reshape(1, -1),
        "ln_g": hw["ln_g"].reshape(1, -1), "ln_b": hw["ln_b"].reshape(1, -1),
        "wh1": hw["h1"]["W"], "bh1": hw["h1"]["b"].reshape(1, -1),
    }
    return _tc_pool_head(h, qs, pool_ws)
